# Initial kernel scaffold; baseline (speedup 1.0000x reference)
import jax
import jax.numpy as jnp
from jax.experimental import pallas as pl


def kernel(hidden_states, router_w, gate_up_proj, down_proj, shared_gate_w, shared_up_w, shared_down_w):
    raise NotImplementedError("write your pallas kernel here")


# speedup vs baseline: 1.0731x; 1.0731x over previous
import functools

import jax
import jax.numpy as jnp
from jax import lax
from jax.experimental import pallas as pl
from jax.experimental.pallas import tpu as pltpu
from jax.experimental.pallas import tpu_sc as plsc

T = 2048
D = 768
FF = 1024
E = 8
K = 2
NP = T * K
BM = 128
P_ROUTED = NP + E * BM
P_TOTAL = P_ROUTED + T
NB_ROUTED = P_ROUTED // BM
NB_TOTAL = P_TOTAL // BM
NW = 32


def _router_body(h_ref, rw_ref, logits_ref, dest_ref, spair_ref, bexp_ref,
                 nv_ref, oh_ref):
    h = h_ref[...]
    rw = rw_ref[...]
    logits = lax.dot_general(h, rw, (((1,), (1,)), ((), ())),
                             preferred_element_type=jnp.float32)
    logits_ref[...] = logits

    ecols = lax.broadcasted_iota(jnp.int32, (T, E), 1)
    m1 = jnp.max(logits, axis=1, keepdims=True)
    e1 = jnp.min(jnp.where(logits == m1, ecols, E), axis=1, keepdims=True)
    masked = jnp.where(ecols == e1, -jnp.inf, logits)
    m2 = jnp.max(masked, axis=1, keepdims=True)
    e2 = jnp.min(jnp.where(masked == m2, ecols, E), axis=1, keepdims=True)

    oh1 = (ecols == e1).astype(jnp.float32)
    oh2 = (ecols == e2).astype(jnp.float32)
    oh_ref[0:T, :] = oh1
    oh_ref[T:NP, :] = oh2

    spair_ref[0:T, :] = jnp.broadcast_to(jax.nn.sigmoid(m1), (T, 16))
    spair_ref[T:NP, :] = jnp.broadcast_to(jax.nn.sigmoid(m2), (T, 16))

    counts = jnp.sum(oh1, axis=0, keepdims=True) + jnp.sum(oh2, axis=0, keepdims=True)
    aligned = jnp.floor((counts + (BM - 1)) / BM) * BM
    ii = lax.broadcasted_iota(jnp.int32, (E, E), 0)
    jj = lax.broadcasted_iota(jnp.int32, (E, E), 1)
    upper = (ii < jj).astype(jnp.float32)
    off = lax.dot_general(aligned, upper, (((1,), (0,)), ((), ())),
                          preferred_element_type=jnp.float32)
    total = jnp.max(off + aligned, axis=1, keepdims=True)
    nv_ref[...] = (total / BM).astype(jnp.int32)

    ri = lax.broadcasted_iota(jnp.int32, (BM, BM), 0)
    rj = lax.broadcasted_iota(jnp.int32, (BM, BM), 1)
    tri = (ri >= rj).astype(jnp.float32)

    def chunk(i, carry):
        oh_c = oh_ref[pl.ds(i * BM, BM), :]
        csum = lax.dot_general(tri, oh_c, (((1,), (0,)), ((), ())),
                               preferred_element_type=jnp.float32)
        tot = carry + csum
        rank = jnp.sum((tot - 1.0) * oh_c, axis=1, keepdims=True)
        offsel = jnp.sum(off * oh_c, axis=1, keepdims=True)
        dest_ref[pl.ds(i * BM, BM), :] = (rank + offsel).astype(jnp.int32)
        return carry + csum[BM - 1:BM, :]

    lax.fori_loop(0, NP // BM, chunk, jnp.zeros((1, E), jnp.float32))

    bv = lax.broadcasted_iota(jnp.int32, (NB_ROUTED, 1), 0).astype(jnp.float32) * BM
    rb = jnp.minimum(bv, total - BM)
    bexp_ref[...] = (jnp.sum((off <= rb).astype(jnp.float32), axis=1,
                             keepdims=True) - 1.0).astype(jnp.int32)


def _router_plan(h, router_w):
    return pl.pallas_call(
        _router_body,
        out_shape=[
            jax.ShapeDtypeStruct((T, E), jnp.float32),
            jax.ShapeDtypeStruct((NP, 1), jnp.int32),
            jax.ShapeDtypeStruct((NP, 16), jnp.float32),
            jax.ShapeDtypeStruct((NB_ROUTED, 1), jnp.int32),
            jax.ShapeDtypeStruct((1, 1), jnp.int32),
        ],
        scratch_shapes=[pltpu.VMEM((NP, E), jnp.float32)],
    )(h, router_w)


def _dispatch_body(h_hbm, dest_hbm, spair_hbm, x_hbm, rows_v, idx_v, s_v, sem):
    c = lax.axis_index("c")
    s = lax.axis_index("s")
    w = s * 2 + c
    base = w * (NP // NW)
    tok0 = base - (base >= T).astype(jnp.int32) * T

    pltpu.sync_copy(h_hbm.at[pl.ds(tok0, NP // NW)], rows_v)
    pltpu.sync_copy(dest_hbm.at[pl.ds(base, NP // NW)], idx_v)
    pltpu.sync_copy(spair_hbm.at[pl.ds(base, NP // NW)], s_v)

    def scale_row(r, _):
        sc = s_v[r, :]
        for j in range(D // 16):
            sl = pl.ds(j * 16, 16)
            rows_v[r, sl] = rows_v[r, sl] * sc
        return 0

    lax.fori_loop(0, NP // NW, scale_row, 0)
    pltpu.async_copy(rows_v, x_hbm.at[idx_v], sem).wait()

    tb = w * (T // NW)
    pltpu.sync_copy(h_hbm.at[pl.ds(tb, T // NW)], rows_v.at[pl.ds(0, T // NW)])
    pltpu.sync_copy(rows_v.at[pl.ds(0, T // NW)],
                    x_hbm.at[pl.ds(P_ROUTED + tb, T // NW)])


@functools.cache
def _dispatch():
    return functools.partial(
        pl.kernel,
        mesh=plsc.VectorSubcoreMesh(core_axis_name="c", subcore_axis_name="s"),
        out_type=jax.ShapeDtypeStruct((P_TOTAL, D), jnp.float32),
        scratch_types=[
            pltpu.VMEM((NP // NW, D), jnp.float32),
            pltpu.VMEM((NP // NW,), jnp.int32),
            pltpu.VMEM((NP // NW, 16), jnp.float32),
            pltpu.SemaphoreType.DMA,
        ],
    )(_dispatch_body)


def _mlp_body(bexp_s, nv_s, x_ref, gup_ref, dp_ref, sg_ref, su_ref,
              sd_ref, y_ref):
    b = pl.program_id(0)

    @pl.when(b < nv_s[0])
    def _routed():
        x = x_ref[...]
        gu = jnp.dot(x, gup_ref[0], preferred_element_type=jnp.float32)
        gate = gu[:, :FF]
        up = gu[:, FF:]
        inter = up * (gate * jax.nn.sigmoid(gate))
        y_ref[...] = jnp.dot(inter, dp_ref[0], preferred_element_type=jnp.float32)

    @pl.when(b >= NB_ROUTED)
    def _shared():
        x = x_ref[...]
        gate = lax.dot_general(x, sg_ref[...], (((1,), (1,)), ((), ())),
                               preferred_element_type=jnp.float32)
        up = lax.dot_general(x, su_ref[...], (((1,), (1,)), ((), ())),
                             preferred_element_type=jnp.float32)
        inter = up * (gate * jax.nn.sigmoid(gate))
        y_ref[...] = lax.dot_general(inter, sd_ref[...], (((1,), (1,)), ((), ())),
                                     preferred_element_type=jnp.float32)


def _grouped_mlp(bexp, nv, x, gup, dp, sg, su, sd):
    grid_spec = pltpu.PrefetchScalarGridSpec(
        num_scalar_prefetch=2,
        grid=(NB_TOTAL,),
        in_specs=[
            pl.BlockSpec((BM, D), lambda b, be, nv: (b, 0)),
            pl.BlockSpec((1, D, 2 * FF),
                         lambda b, be, nv: (be[jnp.minimum(b, NB_ROUTED - 1)], 0, 0)),
            pl.BlockSpec((1, FF, D),
                         lambda b, be, nv: (be[jnp.minimum(b, NB_ROUTED - 1)], 0, 0)),
            pl.BlockSpec((FF, D), lambda b, be, nv: (0, 0)),
            pl.BlockSpec((FF, D), lambda b, be, nv: (0, 0)),
            pl.BlockSpec((D, FF), lambda b, be, nv: (0, 0)),
        ],
        out_specs=pl.BlockSpec((BM, D), lambda b, be, nv: (b, 0)),
    )
    return pl.pallas_call(
        _mlp_body,
        grid_spec=grid_spec,
        out_shape=jax.ShapeDtypeStruct((P_TOTAL, D), jnp.float32),
    )(bexp, nv, x, gup, dp, sg, su, sd)


def _combine_body(y_hbm, dest_hbm, out_hbm, idx1_v, idx2_v, acc_v, g_v, sem):
    c = lax.axis_index("c")
    s = lax.axis_index("s")
    w = s * 2 + c
    t0 = w * (T // NW)
    nt = T // NW

    pltpu.sync_copy(y_hbm.at[pl.ds(P_ROUTED + t0, nt)], acc_v)
    pltpu.sync_copy(dest_hbm.at[pl.ds(t0, nt)], idx1_v)
    pltpu.sync_copy(dest_hbm.at[pl.ds(T + t0, nt)], idx2_v)

    def add_rows(i, _):
        for j in range(D // 16):
            sl = pl.ds(j * 16, 16)
            acc_v[i, sl] = acc_v[i, sl] + g_v[i, sl]
        return 0

    pltpu.async_copy(y_hbm.at[idx1_v], g_v, sem).wait()
    lax.fori_loop(0, nt, add_rows, 0)
    pltpu.async_copy(y_hbm.at[idx2_v], g_v, sem).wait()
    lax.fori_loop(0, nt, add_rows, 0)
    pltpu.sync_copy(acc_v, out_hbm.at[pl.ds(t0, nt)])


@functools.cache
def _combine():
    return functools.partial(
        pl.kernel,
        mesh=plsc.VectorSubcoreMesh(core_axis_name="c", subcore_axis_name="s"),
        out_type=jax.ShapeDtypeStruct((T, D), jnp.float32),
        scratch_types=[
            pltpu.VMEM((T // NW,), jnp.int32),
            pltpu.VMEM((T // NW,), jnp.int32),
            pltpu.VMEM((T // NW, D), jnp.float32),
            pltpu.VMEM((T // NW, D), jnp.float32),
            pltpu.SemaphoreType.DMA,
        ],
    )(_combine_body)


def kernel(hidden_states, router_w, gate_up_proj, down_proj, shared_gate_w,
           shared_up_w, shared_down_w):
    h = hidden_states.reshape(T, D)
    logits, dest, spair, bexp, nv = _router_plan(h, router_w)
    dest1 = dest.reshape(NP)
    x = _dispatch()(h, dest1, spair)
    y = _grouped_mlp(bexp.reshape(NB_ROUTED), nv.reshape(1), x,
                     gate_up_proj, down_proj, shared_gate_w, shared_up_w,
                     shared_down_w)
    out = _combine()(y, dest1)
    return out, logits


# speedup vs baseline: 1.0979x; 1.0231x over previous
import functools

import jax
import jax.numpy as jnp
from jax import lax
from jax.experimental import pallas as pl
from jax.experimental.pallas import tpu as pltpu
from jax.experimental.pallas import tpu_sc as plsc

T = 2048
D = 768
FF = 1024
E = 8
K = 2
NP = T * K
BM = 128
P_ROUTED = NP + E * BM
P_TOTAL = P_ROUTED + T
NB_ROUTED = P_ROUTED // BM
NB_TOTAL = P_TOTAL // BM
NW = 32


def _router_body(h_ref, rw_ref, logits_ref, dest_ref, spair_ref, bexp_ref,
                 nv_ref, oh_ref):
    h = h_ref[...]
    rw = rw_ref[...]
    logits = lax.dot_general(h, rw, (((1,), (1,)), ((), ())),
                             preferred_element_type=jnp.float32)
    logits_ref[...] = logits

    ecols = lax.broadcasted_iota(jnp.int32, (T, E), 1)
    m1 = jnp.max(logits, axis=1, keepdims=True)
    e1 = jnp.min(jnp.where(logits == m1, ecols, E), axis=1, keepdims=True)
    masked = jnp.where(ecols == e1, -jnp.inf, logits)
    m2 = jnp.max(masked, axis=1, keepdims=True)
    e2 = jnp.min(jnp.where(masked == m2, ecols, E), axis=1, keepdims=True)

    oh1 = (ecols == e1).astype(jnp.float32)
    oh2 = (ecols == e2).astype(jnp.float32)
    oh_ref[0:T, :] = oh1
    oh_ref[T:NP, :] = oh2

    spair_ref[0:T, :] = jnp.broadcast_to(jax.nn.sigmoid(m1), (T, 16))
    spair_ref[T:NP, :] = jnp.broadcast_to(jax.nn.sigmoid(m2), (T, 16))

    counts = jnp.sum(oh1, axis=0, keepdims=True) + jnp.sum(oh2, axis=0, keepdims=True)
    aligned = jnp.floor((counts + (BM - 1)) / BM) * BM
    ii = lax.broadcasted_iota(jnp.int32, (E, E), 0)
    jj = lax.broadcasted_iota(jnp.int32, (E, E), 1)
    upper = (ii < jj).astype(jnp.float32)
    off = lax.dot_general(aligned, upper, (((1,), (0,)), ((), ())),
                          preferred_element_type=jnp.float32)
    total = jnp.max(off + aligned, axis=1, keepdims=True)
    nv_ref[...] = (total / BM).astype(jnp.int32)

    ri = lax.broadcasted_iota(jnp.int32, (BM, BM), 0)
    rj = lax.broadcasted_iota(jnp.int32, (BM, BM), 1)
    tri = (ri >= rj).astype(jnp.float32)

    def chunk(i, carry):
        oh_c = oh_ref[pl.ds(i * BM, BM), :]
        csum = lax.dot_general(tri, oh_c, (((1,), (0,)), ((), ())),
                               preferred_element_type=jnp.float32)
        tot = carry + csum
        rank = jnp.sum((tot - 1.0) * oh_c, axis=1, keepdims=True)
        offsel = jnp.sum(off * oh_c, axis=1, keepdims=True)
        dest_ref[pl.ds(i * BM, BM), :] = (rank + offsel).astype(jnp.int32)
        return carry + csum[BM - 1:BM, :]

    lax.fori_loop(0, NP // BM, chunk, jnp.zeros((1, E), jnp.float32))

    bv = lax.broadcasted_iota(jnp.int32, (NB_ROUTED, 1), 0).astype(jnp.float32) * BM
    rb = jnp.minimum(bv, total - BM)
    bexp_ref[...] = (jnp.sum((off <= rb).astype(jnp.float32), axis=1,
                             keepdims=True) - 1.0).astype(jnp.int32)


def _router_plan(h, router_w):
    return pl.pallas_call(
        _router_body,
        out_shape=[
            jax.ShapeDtypeStruct((T, E), jnp.float32),
            jax.ShapeDtypeStruct((NP, 1), jnp.int32),
            jax.ShapeDtypeStruct((NP, 16), jnp.float32),
            jax.ShapeDtypeStruct((NB_ROUTED, 1), jnp.int32),
            jax.ShapeDtypeStruct((1, 1), jnp.int32),
        ],
        scratch_shapes=[pltpu.VMEM((NP, E), jnp.float32)],
    )(h, router_w)


def _dispatch_body(h_hbm, dest_hbm, spair_hbm, x_hbm, rows_v, idx_v, s_v, sem):
    c = lax.axis_index("c")
    s = lax.axis_index("s")
    w = s * 2 + c
    base = w * (NP // NW)
    tok0 = base - (base >= T).astype(jnp.int32) * T

    pltpu.sync_copy(h_hbm.at[pl.ds(tok0, NP // NW)], rows_v)
    pltpu.sync_copy(dest_hbm.at[pl.ds(base, NP // NW)], idx_v)
    pltpu.sync_copy(spair_hbm.at[pl.ds(base, NP // NW)], s_v)

    def scale_row(r, _):
        sc = s_v[r, :]
        for j in range(D // 16):
            sl = pl.ds(j * 16, 16)
            rows_v[r, sl] = rows_v[r, sl] * sc
        return 0

    lax.fori_loop(0, NP // NW, scale_row, 0)
    pltpu.async_copy(rows_v, x_hbm.at[idx_v], sem).wait()


@functools.cache
def _dispatch():
    return functools.partial(
        pl.kernel,
        mesh=plsc.VectorSubcoreMesh(core_axis_name="c", subcore_axis_name="s"),
        out_type=jax.ShapeDtypeStruct((P_ROUTED, D), jnp.float32),
        scratch_types=[
            pltpu.VMEM((NP // NW, D), jnp.float32),
            pltpu.VMEM((NP // NW,), jnp.int32),
            pltpu.VMEM((NP // NW, 16), jnp.float32),
            pltpu.SemaphoreType.DMA,
        ],
    )(_dispatch_body)


def _mlp_body(bexp_s, nv_s, x_ref, h_ref, gup_ref, dp_ref, sg_ref, su_ref,
              sd_ref, y_ref):
    b = pl.program_id(0)
    bf = jnp.bfloat16

    @pl.when(b < nv_s[0])
    def _routed():
        x = x_ref[...].astype(bf)
        gu = jnp.dot(x, gup_ref[0].astype(bf), preferred_element_type=jnp.float32)
        gate = gu[:, :FF]
        up = gu[:, FF:]
        inter = (up * (gate * jax.nn.sigmoid(gate))).astype(bf)
        y_ref[...] = jnp.dot(inter, dp_ref[0].astype(bf),
                             preferred_element_type=jnp.float32)

    @pl.when(b >= NB_ROUTED)
    def _shared():
        x = h_ref[...].astype(bf)
        gate = lax.dot_general(x, sg_ref[...].astype(bf), (((1,), (1,)), ((), ())),
                               preferred_element_type=jnp.float32)
        up = lax.dot_general(x, su_ref[...].astype(bf), (((1,), (1,)), ((), ())),
                             preferred_element_type=jnp.float32)
        inter = (up * (gate * jax.nn.sigmoid(gate))).astype(bf)
        y_ref[...] = lax.dot_general(inter, sd_ref[...].astype(bf),
                                     (((1,), (1,)), ((), ())),
                                     preferred_element_type=jnp.float32)


def _grouped_mlp(bexp, nv, x, h, gup, dp, sg, su, sd):
    grid_spec = pltpu.PrefetchScalarGridSpec(
        num_scalar_prefetch=2,
        grid=(NB_TOTAL,),
        in_specs=[
            pl.BlockSpec((BM, D),
                         lambda b, be, nv: (jnp.minimum(b, NB_ROUTED - 1), 0)),
            pl.BlockSpec((BM, D),
                         lambda b, be, nv: (jnp.maximum(b - NB_ROUTED, 0), 0)),
            pl.BlockSpec((1, D, 2 * FF),
                         lambda b, be, nv: (be[jnp.minimum(b, NB_ROUTED - 1)], 0, 0)),
            pl.BlockSpec((1, FF, D),
                         lambda b, be, nv: (be[jnp.minimum(b, NB_ROUTED - 1)], 0, 0)),
            pl.BlockSpec((FF, D), lambda b, be, nv: (0, 0)),
            pl.BlockSpec((FF, D), lambda b, be, nv: (0, 0)),
            pl.BlockSpec((D, FF), lambda b, be, nv: (0, 0)),
        ],
        out_specs=pl.BlockSpec((BM, D), lambda b, be, nv: (b, 0)),
    )
    return pl.pallas_call(
        _mlp_body,
        grid_spec=grid_spec,
        out_shape=jax.ShapeDtypeStruct((P_TOTAL, D), jnp.float32),
    )(bexp, nv, x, h, gup, dp, sg, su, sd)


def _combine_body(y_hbm, dest_hbm, out_hbm, idx1_v, idx2_v, acc_v, g_v, sem):
    c = lax.axis_index("c")
    s = lax.axis_index("s")
    w = s * 2 + c
    t0 = w * (T // NW)
    nt = T // NW

    pltpu.sync_copy(y_hbm.at[pl.ds(P_ROUTED + t0, nt)], acc_v)
    pltpu.sync_copy(dest_hbm.at[pl.ds(t0, nt)], idx1_v)
    pltpu.sync_copy(dest_hbm.at[pl.ds(T + t0, nt)], idx2_v)

    def add_rows(i, _):
        for j in range(D // 16):
            sl = pl.ds(j * 16, 16)
            acc_v[i, sl] = acc_v[i, sl] + g_v[i, sl]
        return 0

    pltpu.async_copy(y_hbm.at[idx1_v], g_v, sem).wait()
    lax.fori_loop(0, nt, add_rows, 0)
    pltpu.async_copy(y_hbm.at[idx2_v], g_v, sem).wait()
    lax.fori_loop(0, nt, add_rows, 0)
    pltpu.sync_copy(acc_v, out_hbm.at[pl.ds(t0, nt)])


@functools.cache
def _combine():
    return functools.partial(
        pl.kernel,
        mesh=plsc.VectorSubcoreMesh(core_axis_name="c", subcore_axis_name="s"),
        out_type=jax.ShapeDtypeStruct((T, D), jnp.float32),
        scratch_types=[
            pltpu.VMEM((T // NW,), jnp.int32),
            pltpu.VMEM((T // NW,), jnp.int32),
            pltpu.VMEM((T // NW, D), jnp.float32),
            pltpu.VMEM((T // NW, D), jnp.float32),
            pltpu.SemaphoreType.DMA,
        ],
    )(_combine_body)


def kernel(hidden_states, router_w, gate_up_proj, down_proj, shared_gate_w,
           shared_up_w, shared_down_w):
    h = hidden_states.reshape(T, D)
    logits, dest, spair, bexp, nv = _router_plan(h, router_w)
    dest1 = dest.reshape(NP)
    x = _dispatch()(h, dest1, spair)
    y = _grouped_mlp(bexp.reshape(NB_ROUTED), nv.reshape(1), x, h,
                     gate_up_proj, down_proj, shared_gate_w, shared_up_w,
                     shared_down_w)
    out = _combine()(y, dest1)
    return out, logits


# speedup vs baseline: 1.1819x; 1.0766x over previous
import functools

import jax
import jax.numpy as jnp
from jax import lax
from jax.experimental import pallas as pl
from jax.experimental.pallas import tpu as pltpu
from jax.experimental.pallas import tpu_sc as plsc

T = 2048
D = 768
FF = 1024
E = 8
K = 2
NP = T * K
BM = 128
P_ROUTED = NP + E * BM
P_TOTAL = P_ROUTED + T
NB_ROUTED = P_ROUTED // BM
NB_TOTAL = P_TOTAL // BM
NW = 32


def _router_body(h_ref, rw_ref, logits_ref, dest_ref, spair_ref, bexp_ref,
                 nv_ref, oh_ref):
    h = h_ref[...]
    rw = rw_ref[...]
    logits = lax.dot_general(h, rw, (((1,), (1,)), ((), ())),
                             preferred_element_type=jnp.float32)
    logits_ref[...] = logits

    ecols = lax.broadcasted_iota(jnp.int32, (T, E), 1)
    m1 = jnp.max(logits, axis=1, keepdims=True)
    e1 = jnp.min(jnp.where(logits == m1, ecols, E), axis=1, keepdims=True)
    masked = jnp.where(ecols == e1, -jnp.inf, logits)
    m2 = jnp.max(masked, axis=1, keepdims=True)
    e2 = jnp.min(jnp.where(masked == m2, ecols, E), axis=1, keepdims=True)

    oh1 = (ecols == e1).astype(jnp.float32)
    oh2 = (ecols == e2).astype(jnp.float32)
    oh_ref[0:T, :] = oh1
    oh_ref[T:NP, :] = oh2

    spair_ref[0:T, :] = jnp.broadcast_to(jax.nn.sigmoid(m1), (T, 16))
    spair_ref[T:NP, :] = jnp.broadcast_to(jax.nn.sigmoid(m2), (T, 16))

    counts = jnp.sum(oh1, axis=0, keepdims=True) + jnp.sum(oh2, axis=0, keepdims=True)
    aligned = jnp.floor((counts + (BM - 1)) / BM) * BM
    ii = lax.broadcasted_iota(jnp.int32, (E, E), 0)
    jj = lax.broadcasted_iota(jnp.int32, (E, E), 1)
    upper = (ii < jj).astype(jnp.float32)
    off = lax.dot_general(aligned, upper, (((1,), (0,)), ((), ())),
                          preferred_element_type=jnp.float32)
    total = jnp.max(off + aligned, axis=1, keepdims=True)
    nv_ref[...] = (total / BM).astype(jnp.int32)

    ri = lax.broadcasted_iota(jnp.int32, (BM, BM), 0)
    rj = lax.broadcasted_iota(jnp.int32, (BM, BM), 1)
    tri = (ri >= rj).astype(jnp.float32)

    def chunk(i, carry):
        oh_c = oh_ref[pl.ds(i * BM, BM), :]
        csum = lax.dot_general(tri, oh_c, (((1,), (0,)), ((), ())),
                               preferred_element_type=jnp.float32)
        tot = carry + csum
        rank = jnp.sum((tot - 1.0) * oh_c, axis=1, keepdims=True)
        offsel = jnp.sum(off * oh_c, axis=1, keepdims=True)
        dest_ref[pl.ds(i * BM, BM), :] = (rank + offsel).astype(jnp.int32)
        return carry + csum[BM - 1:BM, :]

    lax.fori_loop(0, NP // BM, chunk, jnp.zeros((1, E), jnp.float32))

    bv = lax.broadcasted_iota(jnp.int32, (NB_ROUTED, 1), 0).astype(jnp.float32) * BM
    rb = jnp.minimum(bv, total - BM)
    bexp_ref[...] = (jnp.sum((off <= rb).astype(jnp.float32), axis=1,
                             keepdims=True) - 1.0).astype(jnp.int32)


def _router_plan(h, router_w):
    return pl.pallas_call(
        _router_body,
        out_shape=[
            jax.ShapeDtypeStruct((T, E), jnp.float32),
            jax.ShapeDtypeStruct((NP, 1), jnp.int32),
            jax.ShapeDtypeStruct((NP, 16), jnp.float32),
            jax.ShapeDtypeStruct((NB_ROUTED, 1), jnp.int32),
            jax.ShapeDtypeStruct((1, 1), jnp.int32),
        ],
        scratch_shapes=[pltpu.VMEM((NP, E), jnp.float32)],
    )(h, router_w)


def _dispatch_body(h_hbm, dest_hbm, spair_hbm, x_hbm, rows_v, idx_v, s_v,
                   sem_h, sem_i, sem_s, sem_w):
    c = lax.axis_index("c")
    s = lax.axis_index("s")
    w = s * 2 + c
    npw = NP // NW
    base = w * npw
    tok0 = base - (base >= T).astype(jnp.int32) * T

    cp_h = pltpu.async_copy(h_hbm.at[pl.ds(tok0, npw)], rows_v, sem_h)
    cp_i = pltpu.async_copy(dest_hbm.at[w], idx_v, sem_i)
    cp_s = pltpu.async_copy(spair_hbm.at[pl.ds(base, npw)], s_v, sem_s)
    cp_h.wait()
    cp_s.wait()
    cp_i.wait()

    CH = npw // 4

    def scale_row(r, _):
        sc = s_v[r, :]
        for j in range(D // 16):
            sl = pl.ds(j * 16, 16)
            rows_v[r, sl] = rows_v[r, sl] * sc
        return 0

    copies = []
    for r in range(4):
        lax.fori_loop(r * CH, (r + 1) * CH, scale_row, 0)
        copies.append(pltpu.async_copy(
            rows_v.at[pl.ds(r * CH, CH)], x_hbm.at[idx_v.at[r]], sem_w))
    for cp in copies:
        cp.wait()


@functools.cache
def _dispatch():
    return functools.partial(
        pl.kernel,
        mesh=plsc.VectorSubcoreMesh(core_axis_name="c", subcore_axis_name="s"),
        out_type=jax.ShapeDtypeStruct((P_ROUTED, D), jnp.float32),
        scratch_types=[
            pltpu.VMEM((NP // NW, D), jnp.float32),
            pltpu.VMEM((4, NP // NW // 4), jnp.int32),
            pltpu.VMEM((NP // NW, 16), jnp.float32),
            pltpu.SemaphoreType.DMA,
            pltpu.SemaphoreType.DMA,
            pltpu.SemaphoreType.DMA,
            pltpu.SemaphoreType.DMA,
        ],
    )(_dispatch_body)


def _routed_mlp_body(bexp_s, nv_s, x_ref, gup_ref, dp_ref, y_ref):
    b = pl.program_id(0)
    bf = jnp.bfloat16

    @pl.when(b < nv_s[0])
    def _routed():
        x = x_ref[...].astype(bf)
        gu = jnp.dot(x, gup_ref[0].astype(bf), preferred_element_type=jnp.float32)
        gate = gu[:, :FF]
        up = gu[:, FF:]
        inter = (up * (gate * jax.nn.sigmoid(gate))).astype(bf)
        y_ref[...] = jnp.dot(inter, dp_ref[0].astype(bf),
                             preferred_element_type=jnp.float32)


def _routed_mlp(bexp, nv, x, gup, dp):
    grid_spec = pltpu.PrefetchScalarGridSpec(
        num_scalar_prefetch=2,
        grid=(NB_ROUTED,),
        in_specs=[
            pl.BlockSpec((BM, D), lambda b, be, nv: (b, 0)),
            pl.BlockSpec((1, D, 2 * FF), lambda b, be, nv: (be[b], 0, 0)),
            pl.BlockSpec((1, FF, D), lambda b, be, nv: (be[b], 0, 0)),
        ],
        out_specs=pl.BlockSpec((BM, D), lambda b, be, nv: (b, 0)),
    )
    return pl.pallas_call(
        _routed_mlp_body,
        grid_spec=grid_spec,
        out_shape=jax.ShapeDtypeStruct((P_ROUTED, D), jnp.float32),
    )(bexp, nv, x, gup, dp)


def _shared_mlp_body(h_ref, sg_ref, su_ref, sd_ref, y_ref):
    bf = jnp.bfloat16
    x = h_ref[...].astype(bf)
    gate = lax.dot_general(x, sg_ref[...].astype(bf), (((1,), (1,)), ((), ())),
                           preferred_element_type=jnp.float32)
    up = lax.dot_general(x, su_ref[...].astype(bf), (((1,), (1,)), ((), ())),
                         preferred_element_type=jnp.float32)
    inter = (up * (gate * jax.nn.sigmoid(gate))).astype(bf)
    y_ref[...] = lax.dot_general(inter, sd_ref[...].astype(bf),
                                 (((1,), (1,)), ((), ())),
                                 preferred_element_type=jnp.float32)


def _shared_mlp(h, sg, su, sd):
    return pl.pallas_call(
        _shared_mlp_body,
        grid=(T // BM,),
        in_specs=[
            pl.BlockSpec((BM, D), lambda b: (b, 0)),
            pl.BlockSpec((FF, D), lambda b: (0, 0)),
            pl.BlockSpec((FF, D), lambda b: (0, 0)),
            pl.BlockSpec((D, FF), lambda b: (0, 0)),
        ],
        out_specs=pl.BlockSpec((BM, D), lambda b: (b, 0)),
        out_shape=jax.ShapeDtypeStruct((T, D), jnp.float32),
    )(h, sg, su, sd)


_CR = 16


def _combine_body(yr_hbm, ysh_hbm, dest_hbm, out_hbm, idx1_v, idx2_v, buf,
                  obuf, sem_i, sem_r, sem_o):
    c = lax.axis_index("c")
    s = lax.axis_index("s")
    w = s * 2 + c
    nt = T // NW
    t0 = w * nt
    nrounds = nt // _CR

    cp1 = pltpu.async_copy(dest_hbm.at[pl.ds(t0, nt)], idx1_v, sem_i)
    cp2 = pltpu.async_copy(dest_hbm.at[pl.ds(T + t0, nt)], idx2_v, sem_i)
    cp1.wait()
    cp2.wait()

    def issue(r):
        p = r % 2
        return [
            pltpu.async_copy(ysh_hbm.at[pl.ds(t0 + r * _CR, _CR)],
                             buf.at[p, pl.ds(0, _CR)], sem_r),
            pltpu.async_copy(yr_hbm.at[idx1_v.at[pl.ds(r * _CR, _CR)]],
                             buf.at[p, pl.ds(_CR, _CR)], sem_r),
            pltpu.async_copy(yr_hbm.at[idx2_v.at[pl.ds(r * _CR, _CR)]],
                             buf.at[p, pl.ds(2 * _CR, _CR)], sem_r),
        ]

    pend = issue(0)
    owrites = [None, None]
    for r in range(nrounds):
        for cp in pend:
            cp.wait()
        if r + 1 < nrounds:
            pend = issue(r + 1)
        p = r % 2
        if owrites[p] is not None:
            owrites[p].wait()

        def add_row(i, _, p=p):
            for j in range(D // 16):
                sl = pl.ds(j * 16, 16)
                obuf[p, i, sl] = (buf[p, i, sl] + buf[p, _CR + i, sl]
                                  + buf[p, 2 * _CR + i, sl])
            return 0

        lax.fori_loop(0, _CR, add_row, 0)
        owrites[p] = pltpu.async_copy(
            obuf.at[p], out_hbm.at[pl.ds(t0 + r * _CR, _CR)], sem_o)
    for ow in owrites:
        if ow is not None:
            ow.wait()


@functools.cache
def _combine():
    return functools.partial(
        pl.kernel,
        mesh=plsc.VectorSubcoreMesh(core_axis_name="c", subcore_axis_name="s"),
        out_type=jax.ShapeDtypeStruct((T, D), jnp.float32),
        scratch_types=[
            pltpu.VMEM((T // NW,), jnp.int32),
            pltpu.VMEM((T // NW,), jnp.int32),
            pltpu.VMEM((2, 3 * _CR, D), jnp.float32),
            pltpu.VMEM((2, _CR, D), jnp.float32),
            pltpu.SemaphoreType.DMA,
            pltpu.SemaphoreType.DMA,
            pltpu.SemaphoreType.DMA,
        ],
    )(_combine_body)


def kernel(hidden_states, router_w, gate_up_proj, down_proj, shared_gate_w,
           shared_up_w, shared_down_w):
    h = hidden_states.reshape(T, D)
    logits, dest, spair, bexp, nv = _router_plan(h, router_w)
    ysh = _shared_mlp(h, shared_gate_w, shared_up_w, shared_down_w)
    x = _dispatch()(h, dest.reshape(NW, 4, NP // NW // 4), spair)
    yr = _routed_mlp(bexp.reshape(NB_ROUTED), nv.reshape(1), x,
                     gate_up_proj, down_proj)
    out = _combine()(yr, ysh, dest.reshape(NP))
    return out, logits


# speedup vs baseline: 1.2787x; 1.0818x over previous
import functools

import jax
import jax.numpy as jnp
from jax import lax
from jax.experimental import pallas as pl
from jax.experimental.pallas import tpu as pltpu
from jax.experimental.pallas import tpu_sc as plsc

T = 2048
D = 768
FF = 1024
E = 8
K = 2
NP = T * K
BM = 128
P_ROUTED = NP + E * BM
P_TOTAL = P_ROUTED + T
NB_ROUTED = P_ROUTED // BM
NB_TOTAL = P_TOTAL // BM
NW = 32


def _router_body(h_ref, rw_ref, logits_ref, dest_ref, spair_ref, bexp_ref,
                 nv_ref):
    h = h_ref[...]
    rw = rw_ref[...]
    logits = lax.dot_general(h, rw, (((1,), (1,)), ((), ())),
                             preferred_element_type=jnp.float32)
    logits_ref[...] = logits
    lt = jnp.transpose(logits)

    erows = lax.broadcasted_iota(jnp.int32, (E, T), 0)
    m1 = jnp.max(lt, axis=0, keepdims=True)
    e1 = jnp.min(jnp.where(lt == m1, erows, E), axis=0, keepdims=True)
    masked = jnp.where(erows == e1, -jnp.inf, lt)
    m2 = jnp.max(masked, axis=0, keepdims=True)
    e2 = jnp.min(jnp.where(masked == m2, erows, E), axis=0, keepdims=True)

    spair_ref[0:T, :] = jnp.broadcast_to(
        jnp.transpose(jax.nn.sigmoid(m1)), (T, 16))
    spair_ref[T:NP, :] = jnp.broadcast_to(
        jnp.transpose(jax.nn.sigmoid(m2)), (T, 16))

    oh = jnp.concatenate(
        [(erows == e1).astype(jnp.float32),
         (erows == e2).astype(jnp.float32)], axis=1)

    counts = jnp.sum(oh, axis=1, keepdims=True)
    aligned = jnp.floor((counts + (BM - 1)) / BM) * BM
    ii = lax.broadcasted_iota(jnp.int32, (E, E), 0)
    jj = lax.broadcasted_iota(jnp.int32, (E, E), 1)
    stri = (ii > jj).astype(jnp.float32)
    off = lax.dot_general(stri, aligned, (((1,), (0,)), ((), ())),
                          preferred_element_type=jnp.float32)
    total = jnp.max(off + aligned, axis=0, keepdims=True)
    nv_ref[...] = (total / BM).astype(jnp.int32)

    cum = oh
    k = 1
    while k < NP:
        cum = cum + jnp.concatenate(
            [jnp.zeros((E, k), jnp.float32), cum[:, :NP - k]], axis=1)
        k *= 2
    rank = jnp.sum(cum * oh, axis=0, keepdims=True) - 1.0
    offsel = jnp.sum(off * oh, axis=0, keepdims=True)
    dest_ref[...] = jnp.reshape((rank + offsel).astype(jnp.int32), (NP,))

    bv = lax.broadcasted_iota(jnp.int32, (1, NB_ROUTED), 1).astype(jnp.float32) * BM
    rb = jnp.minimum(bv, total - BM)
    bexp = jnp.sum((off <= rb).astype(jnp.float32), axis=0, keepdims=True) - 1.0
    bexp_ref[...] = jnp.reshape(bexp.astype(jnp.int32), (NB_ROUTED,))


def _router_plan(h, router_w):
    return pl.pallas_call(
        _router_body,
        out_shape=[
            jax.ShapeDtypeStruct((T, E), jnp.float32),
            jax.ShapeDtypeStruct((NP,), jnp.int32),
            jax.ShapeDtypeStruct((NP, 16), jnp.float32),
            jax.ShapeDtypeStruct((NB_ROUTED,), jnp.int32),
            jax.ShapeDtypeStruct((1, 1), jnp.int32),
        ],
    )(h, router_w)


def _dispatch_body(h_hbm, dest_hbm, spair_hbm, x_hbm, rows_v, idx_v, s_v,
                   sem_h, sem_i, sem_s, sem_w):
    c = lax.axis_index("c")
    s = lax.axis_index("s")
    w = s * 2 + c
    npw = NP // NW
    base = w * npw
    tok0 = base - (base >= T).astype(jnp.int32) * T

    cp_h = pltpu.async_copy(h_hbm.at[pl.ds(tok0, npw)], rows_v, sem_h)
    cp_i = pltpu.async_copy(dest_hbm.at[w], idx_v, sem_i)
    cp_s = pltpu.async_copy(spair_hbm.at[pl.ds(base, npw)], s_v, sem_s)
    cp_h.wait()
    cp_s.wait()
    cp_i.wait()

    CH = npw // 4

    def scale_row(r, _):
        sc = s_v[r, :]
        for j in range(D // 16):
            sl = pl.ds(j * 16, 16)
            rows_v[r, sl] = rows_v[r, sl] * sc
        return 0

    copies = []
    for r in range(4):
        lax.fori_loop(r * CH, (r + 1) * CH, scale_row, 0)
        copies.append(pltpu.async_copy(
            rows_v.at[pl.ds(r * CH, CH)], x_hbm.at[idx_v.at[r]], sem_w))
    for cp in copies:
        cp.wait()


@functools.cache
def _dispatch():
    return functools.partial(
        pl.kernel,
        mesh=plsc.VectorSubcoreMesh(core_axis_name="c", subcore_axis_name="s"),
        out_type=jax.ShapeDtypeStruct((P_ROUTED, D), jnp.float32),
        scratch_types=[
            pltpu.VMEM((NP // NW, D), jnp.float32),
            pltpu.VMEM((4, NP // NW // 4), jnp.int32),
            pltpu.VMEM((NP // NW, 16), jnp.float32),
            pltpu.SemaphoreType.DMA,
            pltpu.SemaphoreType.DMA,
            pltpu.SemaphoreType.DMA,
            pltpu.SemaphoreType.DMA,
        ],
    )(_dispatch_body)


def _routed_mlp_body(bexp_s, nv_s, x_ref, gup_ref, dp_ref, y_ref):
    b = pl.program_id(0)
    bf = jnp.bfloat16

    @pl.when(b < nv_s[0])
    def _routed():
        x = x_ref[...].astype(bf)
        gu = jnp.dot(x, gup_ref[0].astype(bf), preferred_element_type=jnp.float32)
        gate = gu[:, :FF]
        up = gu[:, FF:]
        inter = (up * (gate * jax.nn.sigmoid(gate))).astype(bf)
        y_ref[...] = jnp.dot(inter, dp_ref[0].astype(bf),
                             preferred_element_type=jnp.float32)


def _routed_mlp(bexp, nv, x, gup, dp):
    grid_spec = pltpu.PrefetchScalarGridSpec(
        num_scalar_prefetch=2,
        grid=(NB_ROUTED,),
        in_specs=[
            pl.BlockSpec((BM, D), lambda b, be, nv: (b, 0)),
            pl.BlockSpec((1, D, 2 * FF), lambda b, be, nv: (be[b], 0, 0)),
            pl.BlockSpec((1, FF, D), lambda b, be, nv: (be[b], 0, 0)),
        ],
        out_specs=pl.BlockSpec((BM, D), lambda b, be, nv: (b, 0)),
    )
    return pl.pallas_call(
        _routed_mlp_body,
        grid_spec=grid_spec,
        out_shape=jax.ShapeDtypeStruct((P_ROUTED, D), jnp.float32),
    )(bexp, nv, x, gup, dp)


def _shared_mlp_body(h_ref, sg_ref, su_ref, sd_ref, y_ref):
    bf = jnp.bfloat16
    x = h_ref[...].astype(bf)
    gate = lax.dot_general(x, sg_ref[...].astype(bf), (((1,), (1,)), ((), ())),
                           preferred_element_type=jnp.float32)
    up = lax.dot_general(x, su_ref[...].astype(bf), (((1,), (1,)), ((), ())),
                         preferred_element_type=jnp.float32)
    inter = (up * (gate * jax.nn.sigmoid(gate))).astype(bf)
    y_ref[...] = lax.dot_general(inter, sd_ref[...].astype(bf),
                                 (((1,), (1,)), ((), ())),
                                 preferred_element_type=jnp.float32)


def _shared_mlp(h, sg, su, sd):
    return pl.pallas_call(
        _shared_mlp_body,
        grid=(T // BM,),
        in_specs=[
            pl.BlockSpec((BM, D), lambda b: (b, 0)),
            pl.BlockSpec((FF, D), lambda b: (0, 0)),
            pl.BlockSpec((FF, D), lambda b: (0, 0)),
            pl.BlockSpec((D, FF), lambda b: (0, 0)),
        ],
        out_specs=pl.BlockSpec((BM, D), lambda b: (b, 0)),
        out_shape=jax.ShapeDtypeStruct((T, D), jnp.float32),
    )(h, sg, su, sd)


_CR = 16


def _combine_body(yr_hbm, ysh_hbm, dest_hbm, out_hbm, idx1_v, idx2_v, buf,
                  obuf, sem_i, sem_r, sem_o):
    c = lax.axis_index("c")
    s = lax.axis_index("s")
    w = s * 2 + c
    nt = T // NW
    t0 = w * nt
    nrounds = nt // _CR

    cp1 = pltpu.async_copy(dest_hbm.at[pl.ds(t0, nt)], idx1_v, sem_i)
    cp2 = pltpu.async_copy(dest_hbm.at[pl.ds(T + t0, nt)], idx2_v, sem_i)
    cp1.wait()
    cp2.wait()

    def issue(r):
        p = r % 2
        return [
            pltpu.async_copy(ysh_hbm.at[pl.ds(t0 + r * _CR, _CR)],
                             buf.at[p, pl.ds(0, _CR)], sem_r),
            pltpu.async_copy(yr_hbm.at[idx1_v.at[pl.ds(r * _CR, _CR)]],
                             buf.at[p, pl.ds(_CR, _CR)], sem_r),
            pltpu.async_copy(yr_hbm.at[idx2_v.at[pl.ds(r * _CR, _CR)]],
                             buf.at[p, pl.ds(2 * _CR, _CR)], sem_r),
        ]

    pend = issue(0)
    owrites = [None, None]
    for r in range(nrounds):
        for cp in pend:
            cp.wait()
        if r + 1 < nrounds:
            pend = issue(r + 1)
        p = r % 2
        if owrites[p] is not None:
            owrites[p].wait()

        def add_row(i, _, p=p):
            for j in range(D // 16):
                sl = pl.ds(j * 16, 16)
                obuf[p, i, sl] = (buf[p, i, sl] + buf[p, _CR + i, sl]
                                  + buf[p, 2 * _CR + i, sl])
            return 0

        lax.fori_loop(0, _CR, add_row, 0)
        owrites[p] = pltpu.async_copy(
            obuf.at[p], out_hbm.at[pl.ds(t0 + r * _CR, _CR)], sem_o)
    for ow in owrites:
        if ow is not None:
            ow.wait()


@functools.cache
def _combine():
    return functools.partial(
        pl.kernel,
        mesh=plsc.VectorSubcoreMesh(core_axis_name="c", subcore_axis_name="s"),
        out_type=jax.ShapeDtypeStruct((T, D), jnp.float32),
        scratch_types=[
            pltpu.VMEM((T // NW,), jnp.int32),
            pltpu.VMEM((T // NW,), jnp.int32),
            pltpu.VMEM((2, 3 * _CR, D), jnp.float32),
            pltpu.VMEM((2, _CR, D), jnp.float32),
            pltpu.SemaphoreType.DMA,
            pltpu.SemaphoreType.DMA,
            pltpu.SemaphoreType.DMA,
        ],
    )(_combine_body)


def kernel(hidden_states, router_w, gate_up_proj, down_proj, shared_gate_w,
           shared_up_w, shared_down_w):
    h = hidden_states.reshape(T, D)
    logits, dest, spair, bexp, nv = _router_plan(h, router_w)
    ysh = _shared_mlp(h, shared_gate_w, shared_up_w, shared_down_w)
    x = _dispatch()(h, dest.reshape(NW, 4, NP // NW // 4), spair)
    yr = _routed_mlp(bexp, nv.reshape(1), x, gate_up_proj, down_proj)
    out = _combine()(yr, ysh, dest)
    return out, logits


# speedup vs baseline: 1.3647x; 1.0673x over previous
import functools

import jax
import jax.numpy as jnp
from jax import lax
from jax.experimental import pallas as pl
from jax.experimental.pallas import tpu as pltpu
from jax.experimental.pallas import tpu_sc as plsc

T = 2048
D = 768
FF = 1024
E = 8
K = 2
NP = T * K
BM = 128
P_ROUTED = NP + E * BM
P_TOTAL = P_ROUTED + T
NB_ROUTED = P_ROUTED // BM
NB_TOTAL = P_TOTAL // BM
NW = 32


def _router_body(h_ref, rw_ref, logits_ref, dest_ref, spair_ref, bexp_ref,
                 nv_ref, isf_ref, df_ref, fe_ref, slot_ref):
    h = h_ref[...]
    rw = rw_ref[...]
    logits = lax.dot_general(h, rw, (((1,), (1,)), ((), ())),
                             preferred_element_type=jnp.float32)
    logits_ref[...] = logits
    lt = jnp.transpose(logits)

    erows = lax.broadcasted_iota(jnp.int32, (E, T), 0)
    m1 = jnp.max(lt, axis=0, keepdims=True)
    e1 = jnp.min(jnp.where(lt == m1, erows, E), axis=0, keepdims=True)
    masked = jnp.where(erows == e1, -jnp.inf, lt)
    m2 = jnp.max(masked, axis=0, keepdims=True)
    e2 = jnp.min(jnp.where(masked == m2, erows, E), axis=0, keepdims=True)

    spair_ref[0:T, :] = jnp.broadcast_to(
        jnp.transpose(jax.nn.sigmoid(m1)), (T, 16))
    spair_ref[T:NP, :] = jnp.broadcast_to(
        jnp.transpose(jax.nn.sigmoid(m2)), (T, 16))

    oh = jnp.concatenate(
        [(erows == e1).astype(jnp.float32),
         (erows == e2).astype(jnp.float32)], axis=1)

    counts = jnp.sum(oh, axis=1, keepdims=True)
    aligned = jnp.floor((counts + (BM - 1)) / BM) * BM
    ii = lax.broadcasted_iota(jnp.int32, (E, E), 0)
    jj = lax.broadcasted_iota(jnp.int32, (E, E), 1)
    stri = (ii > jj).astype(jnp.float32)
    off = lax.dot_general(stri, aligned, (((1,), (0,)), ((), ())),
                          preferred_element_type=jnp.float32)
    total = jnp.max(off + aligned, axis=0, keepdims=True)
    nv_ref[...] = (total / BM).astype(jnp.int32)

    cum = oh
    k = 1
    while k < NP:
        cum = cum + jnp.concatenate(
            [jnp.zeros((E, k), jnp.float32), cum[:, :NP - k]], axis=1)
        k *= 2
    rank = jnp.sum(cum * oh, axis=0, keepdims=True) - 1.0
    offsel = jnp.sum(off * oh, axis=0, keepdims=True)
    dest_ref[...] = jnp.reshape((rank + offsel).astype(jnp.int32), (NP,))

    bv = lax.broadcasted_iota(jnp.int32, (1, NB_ROUTED), 1).astype(jnp.float32) * BM
    rb = jnp.minimum(bv, total - BM)
    bexp = jnp.sum((off <= rb).astype(jnp.float32), axis=0, keepdims=True) - 1.0
    bexp_ref[...] = jnp.reshape(bexp.astype(jnp.int32), (NB_ROUTED,))

    prev = jnp.concatenate([jnp.full((1, 1), -1.0, jnp.float32),
                            bexp[:, :NB_ROUTED - 1]], axis=1)
    tfirst = (bexp != prev).astype(jnp.float32)
    run = tfirst
    k = 1
    while k < NB_ROUTED:
        run = run + jnp.concatenate(
            [jnp.zeros((1, k), jnp.float32), run[:, :NB_ROUTED - k]], axis=1)
        k *= 2
    run = run - 1.0
    nrun = jnp.max(run, axis=1, keepdims=True) + 1.0
    isf_ref[...] = jnp.reshape(tfirst.astype(jnp.int32), (NB_ROUTED,))
    dofetch = tfirst * (run + 1.0 < nrun).astype(jnp.float32)
    df_ref[...] = jnp.reshape(dofetch.astype(jnp.int32), (NB_ROUTED,))
    slot_ref[...] = jnp.reshape(
        (run - 2.0 * jnp.floor(run / 2.0)).astype(jnp.int32), (NB_ROUTED,))
    runT = jnp.transpose(run)
    tT = jnp.transpose(tfirst)
    bexpT = jnp.transpose(bexp)
    mnext = ((run + 1.0) == runT).astype(jnp.float32) * tT
    fe = jnp.sum(mnext * bexpT, axis=0, keepdims=True)
    fe_ref[...] = jnp.reshape(fe.astype(jnp.int32), (NB_ROUTED,))


def _router_plan(h, router_w):
    return pl.pallas_call(
        _router_body,
        out_shape=[
            jax.ShapeDtypeStruct((T, E), jnp.float32),
            jax.ShapeDtypeStruct((NP,), jnp.int32),
            jax.ShapeDtypeStruct((NP, 16), jnp.float32),
            jax.ShapeDtypeStruct((NB_ROUTED,), jnp.int32),
            jax.ShapeDtypeStruct((1, 1), jnp.int32),
            jax.ShapeDtypeStruct((NB_ROUTED,), jnp.int32),
            jax.ShapeDtypeStruct((NB_ROUTED,), jnp.int32),
            jax.ShapeDtypeStruct((NB_ROUTED,), jnp.int32),
            jax.ShapeDtypeStruct((NB_ROUTED,), jnp.int32),
        ],
    )(h, router_w)


def _dispatch_body(h_hbm, dest_hbm, spair_hbm, x_hbm, rows_v, idx_v, s_v,
                   sem_h, sem_i, sem_s, sem_w):
    c = lax.axis_index("c")
    s = lax.axis_index("s")
    w = s * 2 + c
    npw = NP // NW
    base = w * npw
    tok0 = base - (base >= T).astype(jnp.int32) * T

    CH = npw // 4
    cp_h = pltpu.async_copy(h_hbm.at[pl.ds(tok0, npw)], rows_v, sem_h)
    cps_i = [pltpu.async_copy(dest_hbm.at[pl.ds(base + r * CH, CH)],
                              idx_v.at[r], sem_i) for r in range(4)]
    cp_s = pltpu.async_copy(spair_hbm.at[pl.ds(base, npw)], s_v, sem_s)
    cp_h.wait()
    cp_s.wait()
    for cp in cps_i:
        cp.wait()


    def scale_row(r, _):
        sc = s_v[r, :]
        for j in range(D // 16):
            sl = pl.ds(j * 16, 16)
            rows_v[r, sl] = rows_v[r, sl] * sc
        return 0

    copies = []
    for r in range(4):
        lax.fori_loop(r * CH, (r + 1) * CH, scale_row, 0)
        copies.append(pltpu.async_copy(
            rows_v.at[pl.ds(r * CH, CH)], x_hbm.at[idx_v.at[r]], sem_w))
    for cp in copies:
        cp.wait()


@functools.cache
def _dispatch():
    return functools.partial(
        pl.kernel,
        mesh=plsc.VectorSubcoreMesh(core_axis_name="c", subcore_axis_name="s"),
        out_type=jax.ShapeDtypeStruct((P_ROUTED, D), jnp.float32),
        scratch_types=[
            pltpu.VMEM((NP // NW, D), jnp.float32),
            pltpu.VMEM((4, NP // NW // 4), jnp.int32),
            pltpu.VMEM((NP // NW, 16), jnp.float32),
            pltpu.SemaphoreType.DMA,
            pltpu.SemaphoreType.DMA,
            pltpu.SemaphoreType.DMA,
            pltpu.SemaphoreType.DMA,
        ],
    )(_dispatch_body)


def _routed_mlp_body(bexp_s, nv_s, isf_s, df_s, fe_s, slot_s, x_ref, gup_hbm,
                     dp_hbm, y_ref, wg, wd, semg, semd):
    b = pl.program_id(0)
    bf = jnp.bfloat16
    slot = slot_s[b]

    @pl.when(b == 0)
    def _prologue():
        pltpu.make_async_copy(gup_hbm.at[bexp_s[0]], wg.at[0], semg.at[0]).start()
        pltpu.make_async_copy(dp_hbm.at[bexp_s[0]], wd.at[0], semd.at[0]).start()

    @pl.when(df_s[b] == 1)
    def _prefetch_next():
        nslot = 1 - slot
        pltpu.make_async_copy(gup_hbm.at[fe_s[b]], wg.at[nslot],
                              semg.at[nslot]).start()
        pltpu.make_async_copy(dp_hbm.at[fe_s[b]], wd.at[nslot],
                              semd.at[nslot]).start()

    @pl.when(isf_s[b] == 1)
    def _wait_current():
        pltpu.make_async_copy(gup_hbm.at[bexp_s[b]], wg.at[slot],
                              semg.at[slot]).wait()
        pltpu.make_async_copy(dp_hbm.at[bexp_s[b]], wd.at[slot],
                              semd.at[slot]).wait()

    @pl.when(b < nv_s[0])
    def _routed():
        x = x_ref[...].astype(bf)
        gu = jnp.dot(x, wg[slot].astype(bf), preferred_element_type=jnp.float32)
        gate = gu[:, :FF]
        up = gu[:, FF:]
        inter = (up * (gate * jax.nn.sigmoid(gate))).astype(bf)
        y_ref[...] = jnp.dot(inter, wd[slot].astype(bf),
                             preferred_element_type=jnp.float32)


def _routed_mlp(bexp, nv, isf, df, fe, slot, x, gup, dp):
    grid_spec = pltpu.PrefetchScalarGridSpec(
        num_scalar_prefetch=6,
        grid=(NB_ROUTED,),
        in_specs=[
            pl.BlockSpec((BM, D), lambda b, *s: (b, 0)),
            pl.BlockSpec(memory_space=pl.ANY),
            pl.BlockSpec(memory_space=pl.ANY),
        ],
        out_specs=pl.BlockSpec((BM, D), lambda b, *s: (b, 0)),
        scratch_shapes=[
            pltpu.VMEM((2, D, 2 * FF), jnp.float32),
            pltpu.VMEM((2, FF, D), jnp.float32),
            pltpu.SemaphoreType.DMA((2,)),
            pltpu.SemaphoreType.DMA((2,)),
        ],
    )
    return pl.pallas_call(
        _routed_mlp_body,
        grid_spec=grid_spec,
        out_shape=jax.ShapeDtypeStruct((P_ROUTED, D), jnp.float32),
    )(bexp, nv, isf, df, fe, slot, x, gup, dp)


def _shared_mlp_body(h_ref, sg_ref, su_ref, sd_ref, y_ref):
    bf = jnp.bfloat16
    x = h_ref[...].astype(bf)
    gate = lax.dot_general(x, sg_ref[...].astype(bf), (((1,), (1,)), ((), ())),
                           preferred_element_type=jnp.float32)
    up = lax.dot_general(x, su_ref[...].astype(bf), (((1,), (1,)), ((), ())),
                         preferred_element_type=jnp.float32)
    inter = (up * (gate * jax.nn.sigmoid(gate))).astype(bf)
    y_ref[...] = lax.dot_general(inter, sd_ref[...].astype(bf),
                                 (((1,), (1,)), ((), ())),
                                 preferred_element_type=jnp.float32)


def _shared_mlp(h, sg, su, sd):
    return pl.pallas_call(
        _shared_mlp_body,
        grid=(T // BM,),
        in_specs=[
            pl.BlockSpec((BM, D), lambda b: (b, 0)),
            pl.BlockSpec((FF, D), lambda b: (0, 0)),
            pl.BlockSpec((FF, D), lambda b: (0, 0)),
            pl.BlockSpec((D, FF), lambda b: (0, 0)),
        ],
        out_specs=pl.BlockSpec((BM, D), lambda b: (b, 0)),
        out_shape=jax.ShapeDtypeStruct((T, D), jnp.float32),
    )(h, sg, su, sd)


_CR = 16


def _combine_body(yr_hbm, ysh_hbm, dest_hbm, out_hbm, idx1_v, idx2_v, buf,
                  obuf, sem_i, sem_r, sem_o):
    c = lax.axis_index("c")
    s = lax.axis_index("s")
    w = s * 2 + c
    nt = T // NW
    t0 = w * nt
    nrounds = nt // _CR

    cp1 = pltpu.async_copy(dest_hbm.at[pl.ds(t0, nt)], idx1_v, sem_i)
    cp2 = pltpu.async_copy(dest_hbm.at[pl.ds(T + t0, nt)], idx2_v, sem_i)
    cp1.wait()
    cp2.wait()

    def issue(r):
        p = r % 2
        return [
            pltpu.async_copy(ysh_hbm.at[pl.ds(t0 + r * _CR, _CR)],
                             buf.at[p, pl.ds(0, _CR)], sem_r),
            pltpu.async_copy(yr_hbm.at[idx1_v.at[pl.ds(r * _CR, _CR)]],
                             buf.at[p, pl.ds(_CR, _CR)], sem_r),
            pltpu.async_copy(yr_hbm.at[idx2_v.at[pl.ds(r * _CR, _CR)]],
                             buf.at[p, pl.ds(2 * _CR, _CR)], sem_r),
        ]

    pend = issue(0)
    owrites = [None, None]
    for r in range(nrounds):
        for cp in pend:
            cp.wait()
        if r + 1 < nrounds:
            pend = issue(r + 1)
        p = r % 2
        if owrites[p] is not None:
            owrites[p].wait()

        def add_row(i, _, p=p):
            for j in range(D // 16):
                sl = pl.ds(j * 16, 16)
                obuf[p, i, sl] = (buf[p, i, sl] + buf[p, _CR + i, sl]
                                  + buf[p, 2 * _CR + i, sl])
            return 0

        lax.fori_loop(0, _CR, add_row, 0)
        owrites[p] = pltpu.async_copy(
            obuf.at[p], out_hbm.at[pl.ds(t0 + r * _CR, _CR)], sem_o)
    for ow in owrites:
        if ow is not None:
            ow.wait()


@functools.cache
def _combine():
    return functools.partial(
        pl.kernel,
        mesh=plsc.VectorSubcoreMesh(core_axis_name="c", subcore_axis_name="s"),
        out_type=jax.ShapeDtypeStruct((T, D), jnp.float32),
        scratch_types=[
            pltpu.VMEM((T // NW,), jnp.int32),
            pltpu.VMEM((T // NW,), jnp.int32),
            pltpu.VMEM((2, 3 * _CR, D), jnp.float32),
            pltpu.VMEM((2, _CR, D), jnp.float32),
            pltpu.SemaphoreType.DMA,
            pltpu.SemaphoreType.DMA,
            pltpu.SemaphoreType.DMA,
        ],
    )(_combine_body)


def kernel(hidden_states, router_w, gate_up_proj, down_proj, shared_gate_w,
           shared_up_w, shared_down_w):
    h = hidden_states.reshape(T, D)
    logits, dest, spair, bexp, nv, isf, df, fe, slot = _router_plan(h, router_w)
    ysh = _shared_mlp(h, shared_gate_w, shared_up_w, shared_down_w)
    x = _dispatch()(h, dest, spair)
    yr = _routed_mlp(bexp, nv.reshape(1), isf, df, fe, slot, x,
                     gate_up_proj, down_proj)
    out = _combine()(yr, ysh, dest)
    return out, logits


# speedup vs baseline: 1.3708x; 1.0045x over previous
import functools

import jax
import jax.numpy as jnp
from jax import lax
from jax.experimental import pallas as pl
from jax.experimental.pallas import tpu as pltpu
from jax.experimental.pallas import tpu_sc as plsc

T = 2048
D = 768
FF = 1024
E = 8
K = 2
NP = T * K
BM = 128
P_ROUTED = NP + E * BM
P_TOTAL = P_ROUTED + T
NB_ROUTED = P_ROUTED // BM
NB_TOTAL = P_TOTAL // BM
NW = 32


def _router_body(h_ref, rw_ref, logits_ref, dest_ref, spair_ref, bexp_ref,
                 nv_ref, isf_ref, df_ref, fe_ref, slot_ref):
    h = h_ref[...]
    rw = rw_ref[...]
    logits = lax.dot_general(h, rw, (((1,), (1,)), ((), ())),
                             preferred_element_type=jnp.float32)
    logits_ref[...] = logits
    lt = jnp.transpose(logits)

    erows = lax.broadcasted_iota(jnp.int32, (E, T), 0)
    m1 = jnp.max(lt, axis=0, keepdims=True)
    e1 = jnp.min(jnp.where(lt == m1, erows, E), axis=0, keepdims=True)
    masked = jnp.where(erows == e1, -jnp.inf, lt)
    m2 = jnp.max(masked, axis=0, keepdims=True)
    e2 = jnp.min(jnp.where(masked == m2, erows, E), axis=0, keepdims=True)

    spair_ref[0:T, :] = jnp.broadcast_to(
        jnp.transpose(jax.nn.sigmoid(m1)), (T, 16))
    spair_ref[T:NP, :] = jnp.broadcast_to(
        jnp.transpose(jax.nn.sigmoid(m2)), (T, 16))

    oh = jnp.concatenate(
        [(erows == e1).astype(jnp.float32),
         (erows == e2).astype(jnp.float32)], axis=1)

    counts = jnp.sum(oh, axis=1, keepdims=True)
    aligned = jnp.floor((counts + (BM - 1)) / BM) * BM
    ii = lax.broadcasted_iota(jnp.int32, (E, E), 0)
    jj = lax.broadcasted_iota(jnp.int32, (E, E), 1)
    stri = (ii > jj).astype(jnp.float32)
    off = lax.dot_general(stri, aligned, (((1,), (0,)), ((), ())),
                          preferred_element_type=jnp.float32)
    total = jnp.max(off + aligned, axis=0, keepdims=True)
    nv_ref[...] = (total / BM).astype(jnp.int32)

    cum = oh
    k = 1
    while k < NP:
        cum = cum + jnp.concatenate(
            [jnp.zeros((E, k), jnp.float32), cum[:, :NP - k]], axis=1)
        k *= 2
    rank = jnp.sum(cum * oh, axis=0, keepdims=True) - 1.0
    offsel = jnp.sum(off * oh, axis=0, keepdims=True)
    dest_ref[...] = jnp.reshape((rank + offsel).astype(jnp.int32), (NP,))

    bv = lax.broadcasted_iota(jnp.int32, (1, NB_ROUTED), 1).astype(jnp.float32) * BM
    rb = jnp.minimum(bv, total - BM)
    bexp = jnp.sum((off <= rb).astype(jnp.float32), axis=0, keepdims=True) - 1.0
    bexp_ref[...] = jnp.reshape(bexp.astype(jnp.int32), (NB_ROUTED,))

    prev = jnp.concatenate([jnp.full((1, 1), -1.0, jnp.float32),
                            bexp[:, :NB_ROUTED - 1]], axis=1)
    tfirst = (bexp != prev).astype(jnp.float32)
    run = tfirst
    k = 1
    while k < NB_ROUTED:
        run = run + jnp.concatenate(
            [jnp.zeros((1, k), jnp.float32), run[:, :NB_ROUTED - k]], axis=1)
        k *= 2
    run = run - 1.0
    nrun = jnp.max(run, axis=1, keepdims=True) + 1.0
    isf_ref[...] = jnp.reshape(tfirst.astype(jnp.int32), (NB_ROUTED,))
    dofetch = tfirst * (run + 1.0 < nrun).astype(jnp.float32)
    df_ref[...] = jnp.reshape(dofetch.astype(jnp.int32), (NB_ROUTED,))
    slot_ref[...] = jnp.reshape(
        (run - 2.0 * jnp.floor(run / 2.0)).astype(jnp.int32), (NB_ROUTED,))
    runT = jnp.transpose(run)
    tT = jnp.transpose(tfirst)
    bexpT = jnp.transpose(bexp)
    mnext = ((run + 1.0) == runT).astype(jnp.float32) * tT
    fe = jnp.sum(mnext * bexpT, axis=0, keepdims=True)
    fe_ref[...] = jnp.reshape(fe.astype(jnp.int32), (NB_ROUTED,))


def _router_plan(h, router_w):
    return pl.pallas_call(
        _router_body,
        out_shape=[
            jax.ShapeDtypeStruct((T, E), jnp.float32),
            jax.ShapeDtypeStruct((NP,), jnp.int32),
            jax.ShapeDtypeStruct((NP, 16), jnp.float32),
            jax.ShapeDtypeStruct((NB_ROUTED,), jnp.int32),
            jax.ShapeDtypeStruct((1, 1), jnp.int32),
            jax.ShapeDtypeStruct((NB_ROUTED,), jnp.int32),
            jax.ShapeDtypeStruct((NB_ROUTED,), jnp.int32),
            jax.ShapeDtypeStruct((NB_ROUTED,), jnp.int32),
            jax.ShapeDtypeStruct((NB_ROUTED,), jnp.int32),
        ],
    )(h, router_w)


def _dispatch_body(h_hbm, dest_hbm, spair_hbm, x_hbm, rows_v, idx_v, s_v,
                   sem_h, sem_i, sem_s, sem_w):
    c = lax.axis_index("c")
    s = lax.axis_index("s")
    w = s * 2 + c
    npw = NP // NW
    base = w * npw
    tok0 = base - (base >= T).astype(jnp.int32) * T

    CH = npw // 4
    cp_h = pltpu.async_copy(h_hbm.at[pl.ds(tok0, npw)], rows_v, sem_h)
    cps_i = [pltpu.async_copy(dest_hbm.at[pl.ds(base + r * CH, CH)],
                              idx_v.at[r], sem_i) for r in range(4)]
    cp_s = pltpu.async_copy(spair_hbm.at[pl.ds(base, npw)], s_v, sem_s)
    cp_h.wait()
    cp_s.wait()
    for cp in cps_i:
        cp.wait()


    def scale_row(r, _):
        sc = s_v[r, :]
        for j in range(D // 16):
            sl = pl.ds(j * 16, 16)
            rows_v[r, sl] = rows_v[r, sl] * sc
        return 0

    copies = []
    for r in range(4):
        lax.fori_loop(r * CH, (r + 1) * CH, scale_row, 0)
        copies.append(pltpu.async_copy(
            rows_v.at[pl.ds(r * CH, CH)], x_hbm.at[idx_v.at[r]], sem_w))
    for cp in copies:
        cp.wait()


@functools.cache
def _dispatch():
    return functools.partial(
        pl.kernel,
        mesh=plsc.VectorSubcoreMesh(core_axis_name="c", subcore_axis_name="s"),
        out_type=jax.ShapeDtypeStruct((P_ROUTED, D), jnp.float32),
        scratch_types=[
            pltpu.VMEM((NP // NW, D), jnp.float32),
            pltpu.VMEM((4, NP // NW // 4), jnp.int32),
            pltpu.VMEM((NP // NW, 16), jnp.float32),
            pltpu.SemaphoreType.DMA,
            pltpu.SemaphoreType.DMA,
            pltpu.SemaphoreType.DMA,
            pltpu.SemaphoreType.DMA,
        ],
    )(_dispatch_body)


def _routed_mlp_body(bexp_s, nv_s, isf_s, df_s, fe_s, slot_s, x_ref, gup_hbm,
                     dp_hbm, y_ref, wg, wd, wgb, wdb, semg, semd):
    b = pl.program_id(0)
    bf = jnp.bfloat16
    slot = slot_s[b]

    @pl.when(b == 0)
    def _prologue():
        pltpu.make_async_copy(gup_hbm.at[bexp_s[0]], wg.at[0], semg.at[0]).start()
        pltpu.make_async_copy(dp_hbm.at[bexp_s[0]], wd.at[0], semd.at[0]).start()

    @pl.when(df_s[b] == 1)
    def _prefetch_next():
        nslot = 1 - slot
        pltpu.make_async_copy(gup_hbm.at[fe_s[b]], wg.at[nslot],
                              semg.at[nslot]).start()
        pltpu.make_async_copy(dp_hbm.at[fe_s[b]], wd.at[nslot],
                              semd.at[nslot]).start()

    @pl.when(isf_s[b] == 1)
    def _wait_current():
        pltpu.make_async_copy(gup_hbm.at[bexp_s[b]], wg.at[slot],
                              semg.at[slot]).wait()
        pltpu.make_async_copy(dp_hbm.at[bexp_s[b]], wd.at[slot],
                              semd.at[slot]).wait()
        wgb[slot] = wg[slot].astype(bf)
        wdb[slot] = wd[slot].astype(bf)

    @pl.when(b < nv_s[0])
    def _routed():
        x = x_ref[...].astype(bf)
        gu = jnp.dot(x, wgb[slot], preferred_element_type=jnp.float32)
        gate = gu[:, :FF]
        up = gu[:, FF:]
        inter = (up * (gate * jax.nn.sigmoid(gate))).astype(bf)
        y_ref[...] = jnp.dot(inter, wdb[slot],
                             preferred_element_type=jnp.float32)


def _routed_mlp(bexp, nv, isf, df, fe, slot, x, gup, dp):
    grid_spec = pltpu.PrefetchScalarGridSpec(
        num_scalar_prefetch=6,
        grid=(NB_ROUTED,),
        in_specs=[
            pl.BlockSpec((BM, D), lambda b, *s: (b, 0)),
            pl.BlockSpec(memory_space=pl.ANY),
            pl.BlockSpec(memory_space=pl.ANY),
        ],
        out_specs=pl.BlockSpec((BM, D), lambda b, *s: (b, 0)),
        scratch_shapes=[
            pltpu.VMEM((2, D, 2 * FF), jnp.float32),
            pltpu.VMEM((2, FF, D), jnp.float32),
            pltpu.VMEM((2, D, 2 * FF), jnp.bfloat16),
            pltpu.VMEM((2, FF, D), jnp.bfloat16),
            pltpu.SemaphoreType.DMA((2,)),
            pltpu.SemaphoreType.DMA((2,)),
        ],
    )
    return pl.pallas_call(
        _routed_mlp_body,
        grid_spec=grid_spec,
        out_shape=jax.ShapeDtypeStruct((P_ROUTED, D), jnp.float32),
    )(bexp, nv, isf, df, fe, slot, x, gup, dp)


def _shared_mlp_body(h_ref, sg_ref, su_ref, sd_ref, y_ref):
    bf = jnp.bfloat16
    x = h_ref[...].astype(bf)
    gate = lax.dot_general(x, sg_ref[...].astype(bf), (((1,), (1,)), ((), ())),
                           preferred_element_type=jnp.float32)
    up = lax.dot_general(x, su_ref[...].astype(bf), (((1,), (1,)), ((), ())),
                         preferred_element_type=jnp.float32)
    inter = (up * (gate * jax.nn.sigmoid(gate))).astype(bf)
    y_ref[...] = lax.dot_general(inter, sd_ref[...].astype(bf),
                                 (((1,), (1,)), ((), ())),
                                 preferred_element_type=jnp.float32)


def _shared_mlp(h, sg, su, sd):
    return pl.pallas_call(
        _shared_mlp_body,
        grid=(T // BM,),
        in_specs=[
            pl.BlockSpec((BM, D), lambda b: (b, 0)),
            pl.BlockSpec((FF, D), lambda b: (0, 0)),
            pl.BlockSpec((FF, D), lambda b: (0, 0)),
            pl.BlockSpec((D, FF), lambda b: (0, 0)),
        ],
        out_specs=pl.BlockSpec((BM, D), lambda b: (b, 0)),
        out_shape=jax.ShapeDtypeStruct((T, D), jnp.float32),
    )(h, sg, su, sd)


_CR = 16


def _combine_body(yr_hbm, ysh_hbm, dest_hbm, out_hbm, idx1_v, idx2_v, buf,
                  obuf, sem_i, sem_r, sem_o):
    c = lax.axis_index("c")
    s = lax.axis_index("s")
    w = s * 2 + c
    nt = T // NW
    t0 = w * nt
    nrounds = nt // _CR

    cp1 = pltpu.async_copy(dest_hbm.at[pl.ds(t0, nt)], idx1_v, sem_i)
    cp2 = pltpu.async_copy(dest_hbm.at[pl.ds(T + t0, nt)], idx2_v, sem_i)
    cp1.wait()
    cp2.wait()

    def issue(r):
        p = r % 2
        return [
            pltpu.async_copy(ysh_hbm.at[pl.ds(t0 + r * _CR, _CR)],
                             buf.at[p, pl.ds(0, _CR)], sem_r),
            pltpu.async_copy(yr_hbm.at[idx1_v.at[pl.ds(r * _CR, _CR)]],
                             buf.at[p, pl.ds(_CR, _CR)], sem_r),
            pltpu.async_copy(yr_hbm.at[idx2_v.at[pl.ds(r * _CR, _CR)]],
                             buf.at[p, pl.ds(2 * _CR, _CR)], sem_r),
        ]

    pend = issue(0)
    owrites = [None, None]
    for r in range(nrounds):
        for cp in pend:
            cp.wait()
        if r + 1 < nrounds:
            pend = issue(r + 1)
        p = r % 2
        if owrites[p] is not None:
            owrites[p].wait()

        def add_row(i, _, p=p):
            for j in range(D // 16):
                sl = pl.ds(j * 16, 16)
                obuf[p, i, sl] = (buf[p, i, sl] + buf[p, _CR + i, sl]
                                  + buf[p, 2 * _CR + i, sl])
            return 0

        lax.fori_loop(0, _CR, add_row, 0)
        owrites[p] = pltpu.async_copy(
            obuf.at[p], out_hbm.at[pl.ds(t0 + r * _CR, _CR)], sem_o)
    for ow in owrites:
        if ow is not None:
            ow.wait()


@functools.cache
def _combine():
    return functools.partial(
        pl.kernel,
        mesh=plsc.VectorSubcoreMesh(core_axis_name="c", subcore_axis_name="s"),
        out_type=jax.ShapeDtypeStruct((T, D), jnp.float32),
        scratch_types=[
            pltpu.VMEM((T // NW,), jnp.int32),
            pltpu.VMEM((T // NW,), jnp.int32),
            pltpu.VMEM((2, 3 * _CR, D), jnp.float32),
            pltpu.VMEM((2, _CR, D), jnp.float32),
            pltpu.SemaphoreType.DMA,
            pltpu.SemaphoreType.DMA,
            pltpu.SemaphoreType.DMA,
        ],
    )(_combine_body)


def kernel(hidden_states, router_w, gate_up_proj, down_proj, shared_gate_w,
           shared_up_w, shared_down_w):
    h = hidden_states.reshape(T, D)
    logits, dest, spair, bexp, nv, isf, df, fe, slot = _router_plan(h, router_w)
    ysh = _shared_mlp(h, shared_gate_w, shared_up_w, shared_down_w)
    x = _dispatch()(h, dest, spair)
    yr = _routed_mlp(bexp, nv.reshape(1), isf, df, fe, slot, x,
                     gate_up_proj, down_proj)
    out = _combine()(yr, ysh, dest)
    return out, logits


# speedup vs baseline: 1.5186x; 1.1078x over previous
import functools

import jax
import jax.numpy as jnp
from jax import lax
from jax.experimental import pallas as pl
from jax.experimental.pallas import tpu as pltpu
from jax.experimental.pallas import tpu_sc as plsc

T = 2048
D = 768
FF = 1024
E = 8
K = 2
NP = T * K
BM = 128
P_ROUTED = NP + E * BM
P_TOTAL = P_ROUTED + T
NB_ROUTED = P_ROUTED // BM
NB_TOTAL = P_TOTAL // BM
NW = 32
DP = 1024


def _router_body(h_ref, rw_ref, logits_ref, dest_ref, spair_ref, bexp_ref,
                 nv_ref, isf_ref, df_ref, fe_ref, slot_ref):
    h = h_ref[...]
    rw = rw_ref[...]
    logits = lax.dot_general(h, rw, (((1,), (1,)), ((), ())),
                             preferred_element_type=jnp.float32)
    logits_ref[...] = logits
    lt = jnp.transpose(logits)

    erows = lax.broadcasted_iota(jnp.int32, (E, T), 0)
    m1 = jnp.max(lt, axis=0, keepdims=True)
    e1 = jnp.min(jnp.where(lt == m1, erows, E), axis=0, keepdims=True)
    masked = jnp.where(erows == e1, -jnp.inf, lt)
    m2 = jnp.max(masked, axis=0, keepdims=True)
    e2 = jnp.min(jnp.where(masked == m2, erows, E), axis=0, keepdims=True)

    spair_ref[0:T, :] = jnp.broadcast_to(
        jnp.transpose(jax.nn.sigmoid(m1)), (T, 16))
    spair_ref[T:NP, :] = jnp.broadcast_to(
        jnp.transpose(jax.nn.sigmoid(m2)), (T, 16))

    oh = jnp.concatenate(
        [(erows == e1).astype(jnp.float32),
         (erows == e2).astype(jnp.float32)], axis=1)

    counts = jnp.sum(oh, axis=1, keepdims=True)
    aligned = jnp.floor((counts + (BM - 1)) / BM) * BM
    ii = lax.broadcasted_iota(jnp.int32, (E, E), 0)
    jj = lax.broadcasted_iota(jnp.int32, (E, E), 1)
    stri = (ii > jj).astype(jnp.float32)
    off = lax.dot_general(stri, aligned, (((1,), (0,)), ((), ())),
                          preferred_element_type=jnp.float32)
    total = jnp.max(off + aligned, axis=0, keepdims=True)
    nv_ref[...] = (total / BM).astype(jnp.int32)

    cum = oh
    k = 1
    while k < NP:
        cum = cum + jnp.concatenate(
            [jnp.zeros((E, k), jnp.float32), cum[:, :NP - k]], axis=1)
        k *= 2
    rank = jnp.sum(cum * oh, axis=0, keepdims=True) - 1.0
    offsel = jnp.sum(off * oh, axis=0, keepdims=True)
    dest_ref[...] = jnp.reshape((rank + offsel).astype(jnp.int32), (NP,))

    bv = lax.broadcasted_iota(jnp.int32, (1, NB_ROUTED), 1).astype(jnp.float32) * BM
    rb = jnp.minimum(bv, total - BM)
    bexp = jnp.sum((off <= rb).astype(jnp.float32), axis=0, keepdims=True) - 1.0
    bexp_ref[...] = jnp.reshape(bexp.astype(jnp.int32), (NB_ROUTED,))

    prev = jnp.concatenate([jnp.full((1, 1), -1.0, jnp.float32),
                            bexp[:, :NB_ROUTED - 1]], axis=1)
    tfirst = (bexp != prev).astype(jnp.float32)
    run = tfirst
    k = 1
    while k < NB_ROUTED:
        run = run + jnp.concatenate(
            [jnp.zeros((1, k), jnp.float32), run[:, :NB_ROUTED - k]], axis=1)
        k *= 2
    run = run - 1.0
    nrun = jnp.max(run, axis=1, keepdims=True) + 1.0
    isf_ref[...] = jnp.reshape(tfirst.astype(jnp.int32), (NB_ROUTED,))
    dofetch = tfirst * (run + 1.0 < nrun).astype(jnp.float32)
    df_ref[...] = jnp.reshape(dofetch.astype(jnp.int32), (NB_ROUTED,))
    slot_ref[...] = jnp.reshape(
        (run - 2.0 * jnp.floor(run / 2.0)).astype(jnp.int32), (NB_ROUTED,))
    runT = jnp.transpose(run)
    tT = jnp.transpose(tfirst)
    bexpT = jnp.transpose(bexp)
    mnext = ((run + 1.0) == runT).astype(jnp.float32) * tT
    fe = jnp.sum(mnext * bexpT, axis=0, keepdims=True)
    fe_ref[...] = jnp.reshape(fe.astype(jnp.int32), (NB_ROUTED,))


def _router_plan(h, router_w):
    return pl.pallas_call(
        _router_body,
        out_shape=[
            jax.ShapeDtypeStruct((T, E), jnp.float32),
            jax.ShapeDtypeStruct((NP,), jnp.int32),
            jax.ShapeDtypeStruct((NP, 16), jnp.float32),
            jax.ShapeDtypeStruct((NB_ROUTED,), jnp.int32),
            jax.ShapeDtypeStruct((1, 1), jnp.int32),
            jax.ShapeDtypeStruct((NB_ROUTED,), jnp.int32),
            jax.ShapeDtypeStruct((NB_ROUTED,), jnp.int32),
            jax.ShapeDtypeStruct((NB_ROUTED,), jnp.int32),
            jax.ShapeDtypeStruct((NB_ROUTED,), jnp.int32),
        ],
    )(h, router_w)


def _dispatch_body(h_hbm, dest_hbm, spair_hbm, x_hbm, rows_v, idx_v, s_v,
                   sem_h, sem_i, sem_s, sem_w):
    c = lax.axis_index("c")
    s = lax.axis_index("s")
    w = s * 2 + c
    npw = NP // NW
    base = w * npw
    tok0 = base - (base >= T).astype(jnp.int32) * T

    CH = npw // 4
    cp_h = pltpu.async_copy(h_hbm.at[pl.ds(tok0, npw)], rows_v, sem_h)
    cps_i = [pltpu.async_copy(dest_hbm.at[pl.ds(base + r * CH, CH)],
                              idx_v.at[r], sem_i) for r in range(4)]
    cp_s = pltpu.async_copy(spair_hbm.at[pl.ds(base, npw)], s_v, sem_s)
    cp_h.wait()
    cp_s.wait()
    for cp in cps_i:
        cp.wait()


    def scale_row(r, _):
        sc = s_v[r, :]
        for j in range(D // 16):
            sl = pl.ds(j * 16, 16)
            rows_v[r, sl] = rows_v[r, sl] * sc
        return 0

    copies = []
    for r in range(4):
        lax.fori_loop(r * CH, (r + 1) * CH, scale_row, 0)
        copies.append(pltpu.async_copy(
            rows_v.at[pl.ds(r * CH, CH)], x_hbm.at[idx_v.at[r]], sem_w))
    for cp in copies:
        cp.wait()


@functools.cache
def _dispatch():
    return functools.partial(
        pl.kernel,
        mesh=plsc.VectorSubcoreMesh(core_axis_name="c", subcore_axis_name="s"),
        out_type=jax.ShapeDtypeStruct((P_ROUTED, D), jnp.float32),
        scratch_types=[
            pltpu.VMEM((NP // NW, D), jnp.float32),
            pltpu.VMEM((4, NP // NW // 4), jnp.int32),
            pltpu.VMEM((NP // NW, 16), jnp.float32),
            pltpu.SemaphoreType.DMA,
            pltpu.SemaphoreType.DMA,
            pltpu.SemaphoreType.DMA,
            pltpu.SemaphoreType.DMA,
        ],
    )(_dispatch_body)


def _routed_mlp_body(bexp_s, nv_s, isf_s, df_s, fe_s, slot_s, x_ref, gup_hbm,
                     dp_hbm, y_ref, wg, wd, wgb, wdb, semg, semd):
    b = pl.program_id(0)
    bf = jnp.bfloat16
    slot = slot_s[b]

    @pl.when(b == 0)
    def _prologue():
        pltpu.make_async_copy(gup_hbm.at[bexp_s[0]], wg.at[0], semg.at[0]).start()
        pltpu.make_async_copy(dp_hbm.at[bexp_s[0]], wd.at[0], semd.at[0]).start()

    @pl.when(df_s[b] == 1)
    def _prefetch_next():
        nslot = 1 - slot
        pltpu.make_async_copy(gup_hbm.at[fe_s[b]], wg.at[nslot],
                              semg.at[nslot]).start()
        pltpu.make_async_copy(dp_hbm.at[fe_s[b]], wd.at[nslot],
                              semd.at[nslot]).start()

    @pl.when(isf_s[b] == 1)
    def _wait_current():
        pltpu.make_async_copy(gup_hbm.at[bexp_s[b]], wg.at[slot],
                              semg.at[slot]).wait()
        pltpu.make_async_copy(dp_hbm.at[bexp_s[b]], wd.at[slot],
                              semd.at[slot]).wait()
        wgb[slot] = wg[slot].astype(bf)
        wdb[slot] = wd[slot].astype(bf)

    @pl.when(b < nv_s[0])
    def _routed():
        x = x_ref[...].astype(bf)
        gu = jnp.dot(x, wgb[slot], preferred_element_type=jnp.float32)
        gate = gu[:, :FF]
        up = gu[:, FF:]
        inter = (up * (gate * jax.nn.sigmoid(gate))).astype(bf)
        y_ref[...] = jnp.dot(inter, wdb[slot],
                             preferred_element_type=jnp.float32)


def _routed_mlp(bexp, nv, isf, df, fe, slot, x, gup, dp):
    grid_spec = pltpu.PrefetchScalarGridSpec(
        num_scalar_prefetch=6,
        grid=(NB_ROUTED,),
        in_specs=[
            pl.BlockSpec((BM, D),
                         lambda b, be, nv, *s: (jnp.minimum(b, nv[0] - 1), 0)),
            pl.BlockSpec(memory_space=pl.ANY),
            pl.BlockSpec(memory_space=pl.ANY),
        ],
        out_specs=pl.BlockSpec((BM, D), lambda b, *s: (b, 0)),
        scratch_shapes=[
            pltpu.VMEM((2, D, 2 * FF), jnp.float32),
            pltpu.VMEM((2, FF, D), jnp.float32),
            pltpu.VMEM((2, D, 2 * FF), jnp.bfloat16),
            pltpu.VMEM((2, FF, D), jnp.bfloat16),
            pltpu.SemaphoreType.DMA((2,)),
            pltpu.SemaphoreType.DMA((2,)),
        ],
    )
    return pl.pallas_call(
        _routed_mlp_body,
        grid_spec=grid_spec,
        out_shape=jax.ShapeDtypeStruct((P_ROUTED, D), jnp.float32),
    )(bexp, nv, isf, df, fe, slot, x, gup, dp)


def _shared_mlp_body(h_ref, sg_ref, su_ref, sd_ref, y_ref):
    bf = jnp.bfloat16
    x = h_ref[...].astype(bf)
    gate = lax.dot_general(x, sg_ref[...].astype(bf), (((1,), (1,)), ((), ())),
                           preferred_element_type=jnp.float32)
    up = lax.dot_general(x, su_ref[...].astype(bf), (((1,), (1,)), ((), ())),
                         preferred_element_type=jnp.float32)
    inter = (up * (gate * jax.nn.sigmoid(gate))).astype(bf)
    y_ref[...] = lax.dot_general(inter, sd_ref[...].astype(bf),
                                 (((1,), (1,)), ((), ())),
                                 preferred_element_type=jnp.float32)


def _shared_mlp(h, sg, su, sd):
    bs = 2 * BM
    return pl.pallas_call(
        _shared_mlp_body,
        grid=(T // bs,),
        in_specs=[
            pl.BlockSpec((bs, D), lambda b: (b, 0)),
            pl.BlockSpec((FF, D), lambda b: (0, 0)),
            pl.BlockSpec((FF, D), lambda b: (0, 0)),
            pl.BlockSpec((D, FF), lambda b: (0, 0)),
        ],
        out_specs=pl.BlockSpec((bs, D), lambda b: (b, 0)),
        out_shape=jax.ShapeDtypeStruct((T, D), jnp.float32),
    )(h, sg, su, sd)


_CR = 16


def _combine_body(yr_hbm, ysh_hbm, dest_hbm, out_hbm, idx1_v, idx2_v, buf,
                  obuf, sem_i, sem_r, sem_o):
    c = lax.axis_index("c")
    s = lax.axis_index("s")
    w = s * 2 + c
    nt = T // NW
    t0 = w * nt
    nrounds = nt // _CR

    cp1 = pltpu.async_copy(dest_hbm.at[pl.ds(t0, nt)], idx1_v, sem_i)
    cp2 = pltpu.async_copy(dest_hbm.at[pl.ds(T + t0, nt)], idx2_v, sem_i)
    cp1.wait()
    cp2.wait()

    def issue(r):
        p = r % 2
        return [
            pltpu.async_copy(ysh_hbm.at[pl.ds(t0 + r * _CR, _CR)],
                             buf.at[p, pl.ds(0, _CR)], sem_r),
            pltpu.async_copy(yr_hbm.at[idx1_v.at[pl.ds(r * _CR, _CR)]],
                             buf.at[p, pl.ds(_CR, _CR)], sem_r),
            pltpu.async_copy(yr_hbm.at[idx2_v.at[pl.ds(r * _CR, _CR)]],
                             buf.at[p, pl.ds(2 * _CR, _CR)], sem_r),
        ]

    pend = issue(0)
    owrites = [None, None]
    for r in range(nrounds):
        for cp in pend:
            cp.wait()
        if r + 1 < nrounds:
            pend = issue(r + 1)
        p = r % 2
        if owrites[p] is not None:
            owrites[p].wait()

        def add_row(i, _, p=p):
            for j in range(D // 16):
                sl = pl.ds(j * 16, 16)
                obuf[p, i, sl] = (buf[p, i, sl] + buf[p, _CR + i, sl]
                                  + buf[p, 2 * _CR + i, sl])
            return 0

        lax.fori_loop(0, _CR, add_row, 0)
        owrites[p] = pltpu.async_copy(
            obuf.at[p], out_hbm.at[pl.ds(t0 + r * _CR, _CR)], sem_o)
    for ow in owrites:
        if ow is not None:
            ow.wait()


@functools.cache
def _combine():
    return functools.partial(
        pl.kernel,
        mesh=plsc.VectorSubcoreMesh(core_axis_name="c", subcore_axis_name="s"),
        out_type=jax.ShapeDtypeStruct((T, D), jnp.float32),
        scratch_types=[
            pltpu.VMEM((T // NW,), jnp.int32),
            pltpu.VMEM((T // NW,), jnp.int32),
            pltpu.VMEM((2, 3 * _CR, D), jnp.float32),
            pltpu.VMEM((2, _CR, D), jnp.float32),
            pltpu.SemaphoreType.DMA,
            pltpu.SemaphoreType.DMA,
            pltpu.SemaphoreType.DMA,
        ],
    )(_combine_body)


def kernel(hidden_states, router_w, gate_up_proj, down_proj, shared_gate_w,
           shared_up_w, shared_down_w):
    h = hidden_states.reshape(T, D)
    logits, dest, spair, bexp, nv, isf, df, fe, slot = _router_plan(h, router_w)
    ysh = _shared_mlp(h, shared_gate_w, shared_up_w, shared_down_w)
    x = _dispatch()(h, dest, spair)
    yr = _routed_mlp(bexp, nv.reshape(1), isf, df, fe, slot, x,
                     gate_up_proj, down_proj)
    out = _combine()(yr, ysh, dest)
    return out, logits


# speedup vs baseline: 1.5242x; 1.0037x over previous
import functools

import jax
import jax.numpy as jnp
from jax import lax
from jax.experimental import pallas as pl
from jax.experimental.pallas import tpu as pltpu
from jax.experimental.pallas import tpu_sc as plsc

T = 2048
D = 768
FF = 1024
E = 8
K = 2
NP = T * K
BM = 128
P_ROUTED = NP + E * BM
NB_ROUTED = P_ROUTED // BM
NW = 32


def _router_body(h_ref, rw_ref, logits_ref, dest_ref, spair_ref, bexp_ref,
                 nv_ref, isf_ref, df_ref, fe_ref, slot_ref):
    h = h_ref[...]
    rw = rw_ref[...]
    logits = lax.dot_general(h, rw, (((1,), (1,)), ((), ())),
                             preferred_element_type=jnp.float32)
    logits_ref[...] = logits
    lt = jnp.transpose(logits)

    erows = lax.broadcasted_iota(jnp.int32, (E, T), 0)
    m1 = jnp.max(lt, axis=0, keepdims=True)
    e1 = jnp.min(jnp.where(lt == m1, erows, E), axis=0, keepdims=True)
    masked = jnp.where(erows == e1, -jnp.inf, lt)
    m2 = jnp.max(masked, axis=0, keepdims=True)
    e2 = jnp.min(jnp.where(masked == m2, erows, E), axis=0, keepdims=True)

    spair_ref[0:T, :] = jnp.broadcast_to(
        jnp.transpose(jax.nn.sigmoid(m1)), (T, 16))
    spair_ref[T:NP, :] = jnp.broadcast_to(
        jnp.transpose(jax.nn.sigmoid(m2)), (T, 16))

    oh = jnp.concatenate(
        [(erows == e1).astype(jnp.float32),
         (erows == e2).astype(jnp.float32)], axis=1)

    counts = jnp.sum(oh, axis=1, keepdims=True)
    aligned = jnp.floor((counts + (BM - 1)) / BM) * BM
    ii = lax.broadcasted_iota(jnp.int32, (E, E), 0)
    jj = lax.broadcasted_iota(jnp.int32, (E, E), 1)
    stri = (ii > jj).astype(jnp.float32)
    off = lax.dot_general(stri, aligned, (((1,), (0,)), ((), ())),
                          preferred_element_type=jnp.float32)
    total = jnp.max(off + aligned, axis=0, keepdims=True)
    nv_ref[...] = (total / BM).astype(jnp.int32)

    cum = oh
    k = 1
    while k < NP:
        cum = cum + jnp.concatenate(
            [jnp.zeros((E, k), jnp.float32), cum[:, :NP - k]], axis=1)
        k *= 2
    rank = jnp.sum(cum * oh, axis=0, keepdims=True) - 1.0
    offsel = jnp.sum(off * oh, axis=0, keepdims=True)
    dest_ref[...] = jnp.reshape((rank + offsel).astype(jnp.int32), (NP,))

    bv = lax.broadcasted_iota(jnp.int32, (1, NB_ROUTED), 1).astype(jnp.float32) * BM
    rb = jnp.minimum(bv, total - BM)
    bexp = jnp.sum((off <= rb).astype(jnp.float32), axis=0, keepdims=True) - 1.0
    bexp_ref[...] = jnp.reshape(bexp.astype(jnp.int32), (NB_ROUTED,))

    prev = jnp.concatenate([jnp.full((1, 1), -1.0, jnp.float32),
                            bexp[:, :NB_ROUTED - 1]], axis=1)
    tfirst = (bexp != prev).astype(jnp.float32)
    run = tfirst
    k = 1
    while k < NB_ROUTED:
        run = run + jnp.concatenate(
            [jnp.zeros((1, k), jnp.float32), run[:, :NB_ROUTED - k]], axis=1)
        k *= 2
    run = run - 1.0
    nrun = jnp.max(run, axis=1, keepdims=True) + 1.0
    isf_ref[...] = jnp.reshape(tfirst.astype(jnp.int32), (NB_ROUTED,))
    dofetch = tfirst * (run + 1.0 < nrun).astype(jnp.float32)
    df_ref[...] = jnp.reshape(dofetch.astype(jnp.int32), (NB_ROUTED,))
    slot_ref[...] = jnp.reshape(
        (run - 2.0 * jnp.floor(run / 2.0)).astype(jnp.int32), (NB_ROUTED,))
    runT = jnp.transpose(run)
    tT = jnp.transpose(tfirst)
    bexpT = jnp.transpose(bexp)
    mnext = ((run + 1.0) == runT).astype(jnp.float32) * tT
    fe = jnp.sum(mnext * bexpT, axis=0, keepdims=True)
    fe_ref[...] = jnp.reshape(fe.astype(jnp.int32), (NB_ROUTED,))


def _router_plan(h, router_w):
    return pl.pallas_call(
        _router_body,
        out_shape=[
            jax.ShapeDtypeStruct((T, E), jnp.float32),
            jax.ShapeDtypeStruct((NP,), jnp.int32),
            jax.ShapeDtypeStruct((NP, 16), jnp.float32),
            jax.ShapeDtypeStruct((NB_ROUTED,), jnp.int32),
            jax.ShapeDtypeStruct((1, 1), jnp.int32),
            jax.ShapeDtypeStruct((NB_ROUTED,), jnp.int32),
            jax.ShapeDtypeStruct((NB_ROUTED,), jnp.int32),
            jax.ShapeDtypeStruct((NB_ROUTED,), jnp.int32),
            jax.ShapeDtypeStruct((NB_ROUTED,), jnp.int32),
        ],
    )(h, router_w)


def _dispatch_body(h_hbm, dest_hbm, spair_hbm, x_hbm, rows_v, idx_v, s_v,
                   sem_h, sem_i, sem_s, sem_w):
    c = lax.axis_index("c")
    s = lax.axis_index("s")
    w = s * 2 + c
    npw = NP // NW
    base = w * npw
    tok0 = base - (base >= T).astype(jnp.int32) * T

    CH = npw // 4
    cp_h = pltpu.async_copy(h_hbm.at[pl.ds(tok0, npw)], rows_v, sem_h)
    cps_i = [pltpu.async_copy(dest_hbm.at[pl.ds(base + r * CH, CH)],
                              idx_v.at[r], sem_i) for r in range(4)]
    cp_s = pltpu.async_copy(spair_hbm.at[pl.ds(base, npw)], s_v, sem_s)
    cp_h.wait()
    cp_s.wait()
    for cp in cps_i:
        cp.wait()


    def scale_row(r, _):
        sc = s_v[r, :]
        for j in range(D // 16):
            sl = pl.ds(j * 16, 16)
            rows_v[r, sl] = rows_v[r, sl] * sc
        return 0

    copies = []
    for r in range(4):
        lax.fori_loop(r * CH, (r + 1) * CH, scale_row, 0)
        copies.append(pltpu.async_copy(
            rows_v.at[pl.ds(r * CH, CH)], x_hbm.at[idx_v.at[r]], sem_w))
    for cp in copies:
        cp.wait()


@functools.cache
def _dispatch():
    return functools.partial(
        pl.kernel,
        mesh=plsc.VectorSubcoreMesh(core_axis_name="c", subcore_axis_name="s"),
        out_type=jax.ShapeDtypeStruct((P_ROUTED, D), jnp.float32),
        scratch_types=[
            pltpu.VMEM((NP // NW, D), jnp.float32),
            pltpu.VMEM((4, NP // NW // 4), jnp.int32),
            pltpu.VMEM((NP // NW, 16), jnp.float32),
            pltpu.SemaphoreType.DMA,
            pltpu.SemaphoreType.DMA,
            pltpu.SemaphoreType.DMA,
            pltpu.SemaphoreType.DMA,
        ],
    )(_dispatch_body)


def _routed_mlp_body(bexp_s, nv_s, isf_s, df_s, fe_s, slot_s, x_ref, gup_hbm,
                     dp_hbm, y_ref, wg, wd, wgb, wdb, semg, semd):
    b = pl.program_id(0)
    bf = jnp.bfloat16
    slot = slot_s[b]

    @pl.when(b == 0)
    def _prologue():
        pltpu.make_async_copy(gup_hbm.at[bexp_s[0]], wg.at[0], semg.at[0]).start()
        pltpu.make_async_copy(dp_hbm.at[bexp_s[0]], wd.at[0], semd.at[0]).start()

    @pl.when(df_s[b] == 1)
    def _prefetch_next():
        nslot = 1 - slot
        pltpu.make_async_copy(gup_hbm.at[fe_s[b]], wg.at[nslot],
                              semg.at[nslot]).start()
        pltpu.make_async_copy(dp_hbm.at[fe_s[b]], wd.at[nslot],
                              semd.at[nslot]).start()

    @pl.when(isf_s[b] == 1)
    def _wait_current():
        pltpu.make_async_copy(gup_hbm.at[bexp_s[b]], wg.at[slot],
                              semg.at[slot]).wait()
        pltpu.make_async_copy(dp_hbm.at[bexp_s[b]], wd.at[slot],
                              semd.at[slot]).wait()
        wgb[slot] = wg[slot].astype(bf)
        wdb[slot] = wd[slot].astype(bf)

    @pl.when(b < nv_s[0])
    def _routed():
        x = x_ref[...].astype(bf)
        gu = jnp.dot(x, wgb[slot], preferred_element_type=jnp.float32)
        gate = gu[:, :FF]
        up = gu[:, FF:]
        inter = (up * (gate * jax.nn.sigmoid(gate))).astype(bf)
        y_ref[...] = jnp.dot(inter, wdb[slot],
                             preferred_element_type=jnp.float32)


def _routed_mlp(bexp, nv, isf, df, fe, slot, x, gup, dp):
    grid_spec = pltpu.PrefetchScalarGridSpec(
        num_scalar_prefetch=6,
        grid=(NB_ROUTED,),
        in_specs=[
            pl.BlockSpec((BM, D),
                         lambda b, be, nv, *s: (jnp.minimum(b, nv[0] - 1), 0)),
            pl.BlockSpec(memory_space=pl.ANY),
            pl.BlockSpec(memory_space=pl.ANY),
        ],
        out_specs=pl.BlockSpec((BM, D), lambda b, *s: (b, 0)),
        scratch_shapes=[
            pltpu.VMEM((2, D, 2 * FF), jnp.float32),
            pltpu.VMEM((2, FF, D), jnp.float32),
            pltpu.VMEM((2, D, 2 * FF), jnp.bfloat16),
            pltpu.VMEM((2, FF, D), jnp.bfloat16),
            pltpu.SemaphoreType.DMA((2,)),
            pltpu.SemaphoreType.DMA((2,)),
        ],
    )
    return pl.pallas_call(
        _routed_mlp_body,
        grid_spec=grid_spec,
        out_shape=jax.ShapeDtypeStruct((P_ROUTED, D), jnp.float32),
    )(bexp, nv, isf, df, fe, slot, x, gup, dp)


def _shared_mlp_body(h_ref, sg_ref, su_ref, sd_ref, y_ref):
    bf = jnp.bfloat16
    x = h_ref[...].astype(bf)
    gate = lax.dot_general(x, sg_ref[...].astype(bf), (((1,), (1,)), ((), ())),
                           preferred_element_type=jnp.float32)
    up = lax.dot_general(x, su_ref[...].astype(bf), (((1,), (1,)), ((), ())),
                         preferred_element_type=jnp.float32)
    inter = (up * (gate * jax.nn.sigmoid(gate))).astype(bf)
    y_ref[...] = lax.dot_general(inter, sd_ref[...].astype(bf),
                                 (((1,), (1,)), ((), ())),
                                 preferred_element_type=jnp.float32)


def _shared_mlp(h, sg, su, sd):
    bs = 2 * BM
    return pl.pallas_call(
        _shared_mlp_body,
        grid=(T // bs,),
        in_specs=[
            pl.BlockSpec((bs, D), lambda b: (b, 0)),
            pl.BlockSpec((FF, D), lambda b: (0, 0)),
            pl.BlockSpec((FF, D), lambda b: (0, 0)),
            pl.BlockSpec((D, FF), lambda b: (0, 0)),
        ],
        out_specs=pl.BlockSpec((bs, D), lambda b: (b, 0)),
        out_shape=jax.ShapeDtypeStruct((T, D), jnp.float32),
    )(h, sg, su, sd)


_CR = 16


def _combine_body(yr_hbm, ysh_hbm, dest_hbm, out_hbm, idx1_v, idx2_v, buf,
                  obuf, sem_i, sem_r, sem_o):
    c = lax.axis_index("c")
    s = lax.axis_index("s")
    w = s * 2 + c
    nt = T // NW
    t0 = w * nt
    nrounds = nt // _CR

    cp1 = pltpu.async_copy(dest_hbm.at[pl.ds(t0, nt)], idx1_v, sem_i)
    cp2 = pltpu.async_copy(dest_hbm.at[pl.ds(T + t0, nt)], idx2_v, sem_i)
    cp1.wait()
    cp2.wait()

    def issue(r):
        p = r % 2
        return [
            pltpu.async_copy(ysh_hbm.at[pl.ds(t0 + r * _CR, _CR)],
                             buf.at[p, pl.ds(0, _CR)], sem_r),
            pltpu.async_copy(yr_hbm.at[idx1_v.at[pl.ds(r * _CR, _CR)]],
                             buf.at[p, pl.ds(_CR, _CR)], sem_r),
            pltpu.async_copy(yr_hbm.at[idx2_v.at[pl.ds(r * _CR, _CR)]],
                             buf.at[p, pl.ds(2 * _CR, _CR)], sem_r),
        ]

    pend = issue(0)
    owrites = [None, None]
    for r in range(nrounds):
        for cp in pend:
            cp.wait()
        if r + 1 < nrounds:
            pend = issue(r + 1)
        p = r % 2
        if owrites[p] is not None:
            owrites[p].wait()

        def add_row(i, _, p=p):
            for j in range(D // 16):
                sl = pl.ds(j * 16, 16)
                obuf[p, i, sl] = (buf[p, i, sl] + buf[p, _CR + i, sl]
                                  + buf[p, 2 * _CR + i, sl])
            return 0

        lax.fori_loop(0, _CR, add_row, 0)
        owrites[p] = pltpu.async_copy(
            obuf.at[p], out_hbm.at[pl.ds(t0 + r * _CR, _CR)], sem_o)
    for ow in owrites:
        if ow is not None:
            ow.wait()


@functools.cache
def _combine():
    return functools.partial(
        pl.kernel,
        mesh=plsc.VectorSubcoreMesh(core_axis_name="c", subcore_axis_name="s"),
        out_type=jax.ShapeDtypeStruct((T, D), jnp.float32),
        scratch_types=[
            pltpu.VMEM((T // NW,), jnp.int32),
            pltpu.VMEM((T // NW,), jnp.int32),
            pltpu.VMEM((2, 3 * _CR, D), jnp.float32),
            pltpu.VMEM((2, _CR, D), jnp.float32),
            pltpu.SemaphoreType.DMA,
            pltpu.SemaphoreType.DMA,
            pltpu.SemaphoreType.DMA,
        ],
    )(_combine_body)


def kernel(hidden_states, router_w, gate_up_proj, down_proj, shared_gate_w,
           shared_up_w, shared_down_w):
    h = hidden_states.reshape(T, D)
    logits, dest, spair, bexp, nv, isf, df, fe, slot = _router_plan(h, router_w)
    ysh = _shared_mlp(h, shared_gate_w, shared_up_w, shared_down_w)
    x = _dispatch()(h, dest, spair)
    yr = _routed_mlp(bexp, nv.reshape(1), isf, df, fe, slot, x,
                     gate_up_proj, down_proj)
    out = _combine()(yr, ysh, dest)
    return out, logits
